# Initial kernel scaffold; baseline (speedup 1.0000x reference)
import jax
import jax.numpy as jnp
from jax.experimental import pallas as pl


def kernel(params, tree_fnode, tree_fmess, tree_agraph, tree_bgraph, tree_cgraph, roots, graph_fnode, graph_fmess, graph_agraph, graph_bgraph):
    raise NotImplementedError("write your pallas kernel here")


# speedup vs baseline: 1.3222x; 1.3222x over previous
import functools

import jax
import jax.numpy as jnp
from jax import lax
from jax.experimental import pallas as pl
from jax.experimental.pallas import tpu as pltpu, tpu_sc as plsc

H = 128
MAX_POS = 20
ATOM_SIZE = 40
NBOND = 4
NW = 32
BM = 512
F32 = jnp.float32


def _pick_chunk(bpw, d):
    cap = max(8, min(bpw, (128 * 1024) // (d * 4)))
    for c in range(cap - cap % 8, 7, -8):
        if bpw % c == 0:
            return c
    return 8


def _sc_gather(table, idx):
    v, d = table.shape
    b = idx.shape[0]
    bpw = b // NW
    chunk = _pick_chunk(bpw, d)
    nloop = bpw // chunk
    mesh = plsc.VectorSubcoreMesh(core_axis_name="c", subcore_axis_name="s")

    @functools.partial(
        pl.kernel,
        out_type=jax.ShapeDtypeStruct((b, d), F32),
        mesh=mesh,
        scratch_types=[
            pltpu.VMEM((chunk,), jnp.int32),
            pltpu.VMEM((chunk, d), F32),
            pltpu.SemaphoreType.DMA,
        ],
    )
    def k(table_hbm, idx_hbm, out_hbm, idx_v, rows_v, sem):
        wid = lax.axis_index("s") * 2 + lax.axis_index("c")
        base = wid * bpw

        def body(j, carry):
            off = base + j * chunk
            pltpu.sync_copy(idx_hbm.at[pl.ds(off, chunk)], idx_v)
            pltpu.async_copy(table_hbm.at[idx_v], rows_v, sem).wait()
            pltpu.sync_copy(rows_v, out_hbm.at[pl.ds(off, chunk)])
            return carry

        lax.fori_loop(0, nloop, body, 0)

    return k(table, idx)


def _row(d):
    return pl.BlockSpec((BM, d), lambda i: (i, 0))


def _full(shape):
    return pl.BlockSpec(shape, lambda i: tuple(0 for _ in shape))


def _rowmask(x):
    rows = pl.program_id(0) * BM + lax.broadcasted_iota(jnp.int32, (BM, 1), 0)
    return x * (rows != 0).astype(F32)


def _dot(a, b):
    return jnp.dot(a, b, preferred_element_type=F32)


def _sigm(x):
    return jax.nn.sigmoid(x)


def _k_mm(x, w, b, relu=False):
    m, kdim = x.shape
    n = w.shape[1]

    def body(x_ref, w_ref, b_ref, o_ref):
        acc = _dot(x_ref[...], w_ref[...]) + b_ref[...]
        o_ref[...] = jnp.maximum(acc, 0.0) if relu else acc

    return pl.pallas_call(
        body,
        grid=(m // BM,),
        in_specs=[_row(kdim), _full((kdim, n)), _full((1, n))],
        out_specs=_row(n),
        out_shape=jax.ShapeDtypeStruct((m, n), F32),
    )(x, w, b.reshape(1, n))


def _k_xadd(xg, pos, ptab, bond=None, btab=None):
    m = xg.shape[0]
    n = xg.shape[1]
    pw = ptab.shape[0]

    def body(*refs):
        if btab is None:
            xg_ref, pos_ref, ptab_ref, o_ref = refs
        else:
            xg_ref, pos_ref, bond_ref, ptab_ref, btab_ref, o_ref = refs
        acc = xg_ref[...]
        oh_p = (pos_ref[...] == lax.broadcasted_iota(jnp.int32, (BM, pw), 1)
                ).astype(F32)
        acc = acc + _dot(oh_p, ptab_ref[...])
        if btab is not None:
            bw = btab.shape[0]
            oh_b = (bond_ref[...] == lax.broadcasted_iota(jnp.int32, (BM, bw), 1)
                    ).astype(F32)
            acc = acc + _dot(oh_b, btab_ref[...])
        o_ref[...] = acc

    specs = [_row(n), pl.BlockSpec((BM, 1), lambda i: (i, 0))]
    args = [xg, pos.reshape(m, 1)]
    if btab is not None:
        specs.append(pl.BlockSpec((BM, 1), lambda i: (i, 0)))
        args.append(bond.reshape(m, 1))
    specs.append(_full(ptab.shape))
    args.append(ptab)
    if btab is not None:
        specs.append(_full(btab.shape))
        args.append(btab)

    return pl.pallas_call(
        body,
        grid=(m // BM,),
        in_specs=specs,
        out_specs=_row(n),
        out_shape=jax.ShapeDtypeStruct((m, n), F32),
    )(*args)


def _k_d0(x, wfh):
    m = x.shape[0]

    def body(x_ref, wfh_ref, h_ref, c_ref, g_ref):
        xv = x_ref[...]
        i = _sigm(xv[:, 0:H])
        o = _sigm(xv[:, H:2 * H])
        u = jnp.tanh(xv[:, 2 * H:3 * H])
        c = _rowmask(i * u)
        h = _rowmask(o * jnp.tanh(c))
        h_ref[...] = h
        c_ref[...] = c
        g_ref[...] = _dot(h, wfh_ref[...])

    shp = jax.ShapeDtypeStruct((m, H), F32)
    return pl.pallas_call(
        body,
        grid=(m // BM,),
        in_specs=[_row(4 * H), _full((H, H))],
        out_specs=(_row(H), _row(H), _row(H)),
        out_shape=(shp, shp, shp),
    )(x, wfh)


def _k_d1(x, hn, gn, cn, wh3):
    m = x.shape[0]

    def body(x_ref, hn_ref, gn_ref, cn_ref, wh3_ref, h_ref):
        xv = x_ref[...]
        hnv = hn_ref[...]
        gnv = gn_ref[...]
        cnv = cn_ref[...]
        hs = (hnv[:, 0:H] + hnv[:, H:2 * H]
              + hnv[:, 2 * H:3 * H] + hnv[:, 3 * H:4 * H])
        z = _dot(hs, wh3_ref[...])
        i = _sigm(xv[:, 0:H] + z[:, 0:H])
        o = _sigm(xv[:, H:2 * H] + z[:, H:2 * H])
        u = jnp.tanh(xv[:, 2 * H:3 * H] + z[:, 2 * H:3 * H])
        xf = xv[:, 3 * H:4 * H]
        fc = jnp.zeros((BM, H), F32)
        for a in range(4):
            fc = fc + _sigm(xf + gnv[:, a * H:(a + 1) * H]) * cnv[:, a * H:(a + 1) * H]
        c = i * u + fc
        h_ref[...] = _rowmask(o * jnp.tanh(c))

    return pl.pallas_call(
        body,
        grid=(m // BM,),
        in_specs=[_row(4 * H), _row(4 * H), _row(4 * H), _row(4 * H),
                  _full((H, 3 * H))],
        out_specs=_row(H),
        out_shape=jax.ShapeDtypeStruct((m, H), F32),
    )(x, hn, gn, cn, wh3)


def _k_node(p, nn, w, b, na, mask0):
    m = p.shape[0]

    def body(p_ref, nn_ref, w_ref, b_ref, o_ref):
        nnv = nn_ref[...]
        hs = jnp.zeros((BM, H), F32)
        for a in range(na):
            hs = hs + nnv[:, a * H:(a + 1) * H]
        out = jnp.maximum(p_ref[...] + _dot(hs, w_ref[...]) + b_ref[...], 0.0)
        o_ref[...] = _rowmask(out) if mask0 else out

    return pl.pallas_call(
        body,
        grid=(m // BM,),
        in_specs=[_row(H), _row(na * H), _full((H, H)), _full((1, H))],
        out_specs=_row(H),
        out_shape=jax.ShapeDtypeStruct((m, H), F32),
    )(p, nn, w, b.reshape(1, H))


def _k_hc(eg, hi, hb, w1, w2, b):
    m = eg.shape[0]

    def body(eg_ref, hi_ref, hb_ref, w1_ref, w2_ref, b_ref, o_ref):
        o_ref[...] = jnp.maximum(
            eg_ref[...] + _dot(hi_ref[...], w1_ref[...])
            + _dot(hb_ref[...], w2_ref[...]) + b_ref[...], 0.0)

    return pl.pallas_call(
        body,
        grid=(m // BM,),
        in_specs=[_row(H), _row(H), _row(H), _full((H, H)), _full((H, H)),
                  _full((1, H))],
        out_specs=_row(H),
        out_shape=jax.ShapeDtypeStruct((m, H), F32),
    )(eg, hi, hb, w1, w2, b.reshape(1, H))


def _k_root(fr, rn, w1, w2, b):
    nr = fr.shape[0]

    def body(fr_ref, rn_ref, w1_ref, w2_ref, b_ref, o_ref):
        rnv = rn_ref[...]
        hs = (rnv[:, 0:H] + rnv[:, H:2 * H]
              + rnv[:, 2 * H:3 * H] + rnv[:, 3 * H:4 * H])
        o_ref[...] = jnp.tanh(_dot(fr_ref[...], w1_ref[...])
                              + _dot(hs, w2_ref[...]) + b_ref[...])

    return pl.pallas_call(
        body,
        grid=(1,),
        in_specs=[_full((nr, H)), _full((nr, 4 * H)), _full((H, H)),
                  _full((H, H)), _full((1, H))],
        out_specs=_full((nr, H)),
        out_shape=jax.ShapeDtypeStruct((nr, H), F32),
    )(fr, rn, w1, w2, b.reshape(1, H))


def _ceil_to(x, m):
    return (x + m - 1) // m * m


def _pad_rows(x, n):
    return jnp.pad(x, ((0, n - x.shape[0]),) + ((0, 0),) * (x.ndim - 1))


def _pad_idx(idx, n):
    return jnp.pad(idx.astype(jnp.int32), (0, n - idx.shape[0]))


def _gate_weights(p):
    wcat = jnp.concatenate(
        [p['Wi'][0], p['Wog'][0], p['Wu'][0], p['Wf'][0]], axis=1)
    bcat = jnp.concatenate(
        [p['Wi'][1], p['Wog'][1], p['Wu'][1], p['Wf'][1]])
    return wcat, bcat


def _msg_pass(x, bgraph_flat_p, wcat, din, mp):
    wfh = wcat[din:, 3 * H:4 * H]
    wh3 = wcat[din:, 0:3 * H]
    h0, c0, g0 = _k_d0(x, wfh)
    hn = _sc_gather(h0, bgraph_flat_p).reshape(mp, 4 * H)
    gn = _sc_gather(g0, bgraph_flat_p).reshape(mp, 4 * H)
    cn = _sc_gather(c0, bgraph_flat_p).reshape(mp, 4 * H)
    return _k_d1(x, hn, gn, cn, wh3)


def kernel(params, tree_fnode, tree_fmess, tree_agraph, tree_bgraph,
           tree_cgraph, roots, graph_fnode, graph_fmess, graph_agraph,
           graph_bgraph):
    p = params
    nt, mt = tree_fnode.shape[0], tree_fmess.shape[0]
    ng, mg = graph_fnode.shape[0], graph_fmess.shape[0]
    cl = tree_cgraph.shape[1]
    nroot = roots.shape[0]
    ntp, mtp = _ceil_to(nt, 2048), _ceil_to(mt, 2048)
    ngp, mgp = _ceil_to(ng, 2048), _ceil_to(mg, 2048)

    fnode_g_p = _pad_idx(graph_fnode, ngp)
    src_g = _pad_idx(graph_fmess[:, 0], mgp)
    bond_g = _pad_idx(graph_fmess[:, 2], mgp)
    pos_g = _pad_idx(graph_fmess[:, 3], mgp)
    bg_g = _pad_idx(graph_bgraph.reshape(-1), mgp * 4)
    ag_g = _pad_idx(graph_agraph.reshape(-1), ngp * 4)
    src_t = _pad_idx(tree_fmess[:, 0], mtp)
    pos_t = _pad_idx(tree_fmess[:, 2], mtp)
    bg_t = _pad_idx(tree_bgraph.reshape(-1), mtp * 4)
    ag_t = _pad_idx(tree_agraph.reshape(-1), ntp * 4)
    cg_t = _pad_idx(tree_cgraph.reshape(-1), ntp * cl)
    tf0 = _pad_idx(tree_fnode[:, 0], ntp)
    tf1 = _pad_idx(tree_fnode[:, 1], ntp)
    tf2 = _pad_idx(tree_fnode[:, 2], ntp)

    wcat_g, bcat_g = _gate_weights(p['enc_graph'])
    atom_tab = wcat_g[0:ATOM_SIZE]
    bond_tab = _pad_rows(wcat_g[ATOM_SIZE:ATOM_SIZE + NBOND], 8)
    pos_tab_g = _pad_rows(wcat_g[ATOM_SIZE + NBOND:ATOM_SIZE + NBOND + MAX_POS]
                          + bcat_g[None, :], 24)
    atom_rows = _sc_gather(atom_tab, fnode_g_p)
    xg_g = _sc_gather(atom_rows, src_g)
    x_g = _k_xadd(xg_g, pos_g, pos_tab_g, bond_g, bond_tab)
    h1_g = _msg_pass(x_g, bg_g, wcat_g, ATOM_SIZE + NBOND + MAX_POS, mgp)
    wo_g, bo_g = p['enc_graph']['Wo']
    p_g = _sc_gather(wo_g[0:ATOM_SIZE], fnode_g_p)
    nn_g = _sc_gather(h1_g, ag_g).reshape(ngp, 4 * H)
    hatom_p = _k_node(p_g, nn_g, wo_g[ATOM_SIZE:], bo_g, 4, True)

    wcat_b, bcat_b = _gate_weights(p['enc_bond'])
    hnode_b = _sc_gather(p['E_l'], tf2)
    t_b = _k_mm(hnode_b, wcat_b[0:H], bcat_b)
    x_b = _k_xadd(_sc_gather(t_b, src_t), pos_t,
                  _pad_rows(wcat_b[H:H + MAX_POS], 24))
    h1_b = _msg_pass(x_b, bg_t, wcat_b, H + MAX_POS, mtp)
    wo_b, bo_b = p['enc_bond']['Wo']
    p_b = _k_mm(hnode_b, wo_b[0:H], jnp.zeros((H,), F32))
    nn_b = _sc_gather(h1_b, ag_t).reshape(ntp, 4 * H)
    hbond_p = _k_node(p_b, nn_b, wo_b[H:], bo_b, 4, True)

    w_i, b_i = p['W_i']
    ei_tab = _k_mm(_pad_rows(p['E_i'], _ceil_to(p['E_i'].shape[0], BM)),
                   w_i[0:H], jnp.zeros((H,), F32))
    eg_f = _sc_gather(ei_tab, tf1)
    cl_rows = _sc_gather(hatom_p, cg_t).reshape(ntp, cl * H)
    hnode_f = _k_node(eg_f, cl_rows, w_i[H:], b_i, cl, False)
    wcat_f, bcat_f = _gate_weights(p['enc_frag'])
    t_f = _k_mm(hnode_f, wcat_f[0:H], bcat_f)
    x_f = _k_xadd(_sc_gather(t_f, src_t), pos_t,
                  _pad_rows(wcat_f[H:H + MAX_POS], 24))
    h1_f = _msg_pass(x_f, bg_t, wcat_f, H + MAX_POS, mtp)
    wo_f, bo_f = p['enc_frag']['Wo']
    p_f = _k_mm(hnode_f, wo_f[0:H], jnp.zeros((H,), F32))
    nn_f = _sc_gather(h1_f, ag_t).reshape(ntp, 4 * H)
    hinter_p = _k_node(p_f, nn_f, wo_f[H:], bo_f, 4, True)

    w_c, b_c = p['W_c']
    ec_tab = _k_mm(_pad_rows(p['E_c'], _ceil_to(p['E_c'].shape[0], BM)),
                   w_c[0:H], jnp.zeros((H,), F32))
    eg_c = _sc_gather(ec_tab, tf0)
    hnode_c = _k_hc(eg_c, hinter_p, hbond_p, w_c[H:2 * H], w_c[2 * H:], b_c)
    wcat_c, bcat_c = _gate_weights(p['enc_inter'])
    t_c = _k_mm(hnode_c, wcat_c[0:H], bcat_c)
    x_c = _k_xadd(_sc_gather(t_c, src_t), pos_t,
                  _pad_rows(wcat_c[H:H + MAX_POS], 24))
    h1_c = _msg_pass(x_c, bg_t, wcat_c, H + MAX_POS, mtp)
    wo_c, bo_c = p['enc_inter']['Wo']
    p_c = _k_mm(hnode_c, wo_c[0:H], jnp.zeros((H,), F32))
    nn_c = _sc_gather(h1_c, ag_t).reshape(ntp, 4 * H)
    hnode_p = _k_node(p_c, nn_c, wo_c[H:], bo_c, 4, True)

    wr, br = p['W_root']
    roots_p = _pad_idx(roots, 256)
    fr = _sc_gather(hnode_c, roots_p)[:nroot]
    agr = _pad_idx(jnp.take(tree_agraph, roots, axis=0).reshape(-1), 256)
    rn = _sc_gather(h1_c, agr)[:nroot * 4].reshape(nroot, 4 * H)
    hroot = _k_root(fr, rn, wr[0:H], wr[H:], br)

    return (hroot, hnode_p[:nt], hinter_p[:nt], hbond_p[:nt], hatom_p[:ng])


# speedup vs baseline: 1.5271x; 1.1550x over previous
import functools

import jax
import jax.numpy as jnp
from jax import lax
from jax.experimental import pallas as pl
from jax.experimental.pallas import tpu as pltpu, tpu_sc as plsc

H = 128
MAX_POS = 20
ATOM_SIZE = 40
NBOND = 4
NW = 32
BM = 512
F32 = jnp.float32


def _pick_chunk(bpw, d):
    cap = max(8, min(bpw, (128 * 1024) // (d * 4)))
    for c in range(cap - cap % 8, 7, -8):
        if bpw % c == 0:
            return c
    return 8


def _sc_gather(table, idx):
    v, d = table.shape
    b = idx.shape[0]
    bpw = b // NW
    chunk = _pick_chunk(bpw, d)
    nloop = bpw // chunk
    mesh = plsc.VectorSubcoreMesh(core_axis_name="c", subcore_axis_name="s")

    @functools.partial(
        pl.kernel,
        out_type=jax.ShapeDtypeStruct((b, d), F32),
        mesh=mesh,
        scratch_types=[
            pltpu.VMEM((chunk,), jnp.int32),
            pltpu.VMEM((chunk, d), F32),
            pltpu.SemaphoreType.DMA,
        ],
    )
    def k(table_hbm, idx_hbm, out_hbm, idx_v, rows_v, sem):
        wid = lax.axis_index("s") * 2 + lax.axis_index("c")
        base = wid * bpw

        def body(j, carry):
            off = base + j * chunk
            pltpu.sync_copy(idx_hbm.at[pl.ds(off, chunk)], idx_v)
            pltpu.async_copy(table_hbm.at[idx_v], rows_v, sem).wait()
            pltpu.sync_copy(rows_v, out_hbm.at[pl.ds(off, chunk)])
            return carry

        lax.fori_loop(0, nloop, body, 0)

    return k(table, idx)


def _sc_gather_nbr(tables, idx_t):
    a_dim, mp = idx_t.shape
    ntab = len(tables)
    bpw = mp // NW
    chunk = _pick_chunk(bpw, H)
    nloop = bpw // chunk
    mesh = plsc.VectorSubcoreMesh(core_axis_name="c", subcore_axis_name="s")
    shp = jax.ShapeDtypeStruct((a_dim * mp, H), F32)

    @functools.partial(
        pl.kernel,
        out_type=tuple(shp for _ in range(ntab)),
        mesh=mesh,
        scratch_types=[
            pltpu.VMEM((chunk,), jnp.int32),
            pltpu.VMEM((chunk, H), F32),
            pltpu.SemaphoreType.DMA,
        ],
    )
    def k(*refs):
        tab_refs = refs[:ntab]
        idx_hbm = refs[ntab]
        out_refs = refs[ntab + 1:ntab + 1 + ntab]
        idx_v, rows_v, sem = refs[ntab + 1 + ntab:]
        wid = lax.axis_index("s") * 2 + lax.axis_index("c")
        base = wid * bpw

        def body(j, carry):
            off = base + j * chunk
            for a in range(a_dim):
                pltpu.sync_copy(idx_hbm.at[pl.ds(a * mp + off, chunk)], idx_v)
                for t in range(ntab):
                    pltpu.async_copy(tab_refs[t].at[idx_v], rows_v, sem).wait()
                    pltpu.sync_copy(
                        rows_v, out_refs[t].at[pl.ds(a * mp + off, chunk)])
            return carry

        lax.fori_loop(0, nloop, body, 0)

    out = k(*tables, idx_t.reshape(-1))
    out = tuple(out) if isinstance(out, (list, tuple)) else (out,)
    return tuple(o.reshape(a_dim, mp, H) for o in out)


def _row(d):
    return pl.BlockSpec((BM, d), lambda i: (i, 0))


def _full(shape):
    return pl.BlockSpec(shape, lambda i: tuple(0 for _ in shape))


def _rowmask(x):
    rows = pl.program_id(0) * BM + lax.broadcasted_iota(jnp.int32, (BM, 1), 0)
    return x * (rows != 0).astype(F32)


def _dot(a, b):
    return jnp.dot(a, b, preferred_element_type=F32)


def _sigm(x):
    return jax.nn.sigmoid(x)


def _k_mm(x, w, b, relu=False):
    m, kdim = x.shape
    n = w.shape[1]

    def body(x_ref, w_ref, b_ref, o_ref):
        acc = _dot(x_ref[...], w_ref[...]) + b_ref[...]
        o_ref[...] = jnp.maximum(acc, 0.0) if relu else acc

    return pl.pallas_call(
        body,
        grid=(m // BM,),
        in_specs=[_row(kdim), _full((kdim, n)), _full((1, n))],
        out_specs=_row(n),
        out_shape=jax.ShapeDtypeStruct((m, n), F32),
    )(x, w, b.reshape(1, n))


def _k_xadd(xg, pos, ptab, bond=None, btab=None):
    m = xg.shape[0]
    n = xg.shape[1]
    pw = ptab.shape[0]

    def body(*refs):
        if btab is None:
            xg_ref, pos_ref, ptab_ref, o_ref = refs
        else:
            xg_ref, pos_ref, bond_ref, ptab_ref, btab_ref, o_ref = refs
        acc = xg_ref[...]
        oh_p = (pos_ref[...] == lax.broadcasted_iota(jnp.int32, (BM, pw), 1)
                ).astype(F32)
        acc = acc + _dot(oh_p, ptab_ref[...])
        if btab is not None:
            bw = btab.shape[0]
            oh_b = (bond_ref[...] == lax.broadcasted_iota(jnp.int32, (BM, bw), 1)
                    ).astype(F32)
            acc = acc + _dot(oh_b, btab_ref[...])
        o_ref[...] = acc

    specs = [_row(n), pl.BlockSpec((BM, 1), lambda i: (i, 0))]
    args = [xg, pos.reshape(m, 1)]
    if btab is not None:
        specs.append(pl.BlockSpec((BM, 1), lambda i: (i, 0)))
        args.append(bond.reshape(m, 1))
    specs.append(_full(ptab.shape))
    args.append(ptab)
    if btab is not None:
        specs.append(_full(btab.shape))
        args.append(btab)

    return pl.pallas_call(
        body,
        grid=(m // BM,),
        in_specs=specs,
        out_specs=_row(n),
        out_shape=jax.ShapeDtypeStruct((m, n), F32),
    )(*args)


def _k_d0(x, wfh):
    m = x.shape[0]

    def body(x_ref, wfh_ref, h_ref, c_ref, g_ref):
        xv = x_ref[...]
        i = _sigm(xv[:, 0:H])
        o = _sigm(xv[:, H:2 * H])
        u = jnp.tanh(xv[:, 2 * H:3 * H])
        c = _rowmask(i * u)
        h = _rowmask(o * jnp.tanh(c))
        h_ref[...] = h
        c_ref[...] = c
        g_ref[...] = _dot(h, wfh_ref[...])

    shp = jax.ShapeDtypeStruct((m, H), F32)
    return pl.pallas_call(
        body,
        grid=(m // BM,),
        in_specs=[_row(4 * H), _full((H, H))],
        out_specs=(_row(H), _row(H), _row(H)),
        out_shape=(shp, shp, shp),
    )(x, wfh)


def _k_d1(x, hn, gn, cn, wh3):
    m = x.shape[0]

    nbr = pl.BlockSpec((4, BM, H), lambda i: (0, i, 0))

    def body(x_ref, hn_ref, gn_ref, cn_ref, wh3_ref, h_ref):
        xv = x_ref[...]
        hs = (hn_ref[0] + hn_ref[1] + hn_ref[2] + hn_ref[3])
        z = _dot(hs, wh3_ref[...])
        i = _sigm(xv[:, 0:H] + z[:, 0:H])
        o = _sigm(xv[:, H:2 * H] + z[:, H:2 * H])
        u = jnp.tanh(xv[:, 2 * H:3 * H] + z[:, 2 * H:3 * H])
        xf = xv[:, 3 * H:4 * H]
        fc = jnp.zeros((BM, H), F32)
        for a in range(4):
            fc = fc + _sigm(xf + gn_ref[a]) * cn_ref[a]
        c = i * u + fc
        h_ref[...] = _rowmask(o * jnp.tanh(c))

    return pl.pallas_call(
        body,
        grid=(m // BM,),
        in_specs=[_row(4 * H), nbr, nbr, nbr, _full((H, 3 * H))],
        out_specs=_row(H),
        out_shape=jax.ShapeDtypeStruct((m, H), F32),
    )(x, hn, gn, cn, wh3)


def _k_node(p, nn, w, b, na, mask0):
    m = p.shape[0]

    def body(p_ref, nn_ref, w_ref, b_ref, o_ref):
        hs = jnp.zeros((BM, H), F32)
        for a in range(na):
            hs = hs + nn_ref[a]
        out = jnp.maximum(p_ref[...] + _dot(hs, w_ref[...]) + b_ref[...], 0.0)
        o_ref[...] = _rowmask(out) if mask0 else out

    return pl.pallas_call(
        body,
        grid=(m // BM,),
        in_specs=[_row(H), pl.BlockSpec((na, BM, H), lambda i: (0, i, 0)),
                  _full((H, H)), _full((1, H))],
        out_specs=_row(H),
        out_shape=jax.ShapeDtypeStruct((m, H), F32),
    )(p, nn, w, b.reshape(1, H))


def _k_hc(eg, hi, hb, w1, w2, b):
    m = eg.shape[0]

    def body(eg_ref, hi_ref, hb_ref, w1_ref, w2_ref, b_ref, o_ref):
        o_ref[...] = jnp.maximum(
            eg_ref[...] + _dot(hi_ref[...], w1_ref[...])
            + _dot(hb_ref[...], w2_ref[...]) + b_ref[...], 0.0)

    return pl.pallas_call(
        body,
        grid=(m // BM,),
        in_specs=[_row(H), _row(H), _row(H), _full((H, H)), _full((H, H)),
                  _full((1, H))],
        out_specs=_row(H),
        out_shape=jax.ShapeDtypeStruct((m, H), F32),
    )(eg, hi, hb, w1, w2, b.reshape(1, H))


def _k_root(fr, rn, w1, w2, b):
    nr = fr.shape[0]

    def body(fr_ref, rn_ref, w1_ref, w2_ref, b_ref, o_ref):
        rnv = rn_ref[...]
        hs = (rnv[:, 0:H] + rnv[:, H:2 * H]
              + rnv[:, 2 * H:3 * H] + rnv[:, 3 * H:4 * H])
        o_ref[...] = jnp.tanh(_dot(fr_ref[...], w1_ref[...])
                              + _dot(hs, w2_ref[...]) + b_ref[...])

    return pl.pallas_call(
        body,
        grid=(1,),
        in_specs=[_full((nr, H)), _full((nr, 4 * H)), _full((H, H)),
                  _full((H, H)), _full((1, H))],
        out_specs=_full((nr, H)),
        out_shape=jax.ShapeDtypeStruct((nr, H), F32),
    )(fr, rn, w1, w2, b.reshape(1, H))


def _ceil_to(x, m):
    return (x + m - 1) // m * m


def _pad_rows(x, n):
    return jnp.pad(x, ((0, n - x.shape[0]),) + ((0, 0),) * (x.ndim - 1))


def _pad_idx(idx, n):
    return jnp.pad(idx.astype(jnp.int32), (0, n - idx.shape[0]))


def _gate_weights(p):
    wcat = jnp.concatenate(
        [p['Wi'][0], p['Wog'][0], p['Wu'][0], p['Wf'][0]], axis=1)
    bcat = jnp.concatenate(
        [p['Wi'][1], p['Wog'][1], p['Wu'][1], p['Wf'][1]])
    return wcat, bcat


def _pad_idx2(idx, n):
    return jnp.pad(idx.astype(jnp.int32), ((0, 0), (0, n - idx.shape[1])))


def _msg_pass(x, bg_t, wcat, din, mp):
    wfh = wcat[din:, 3 * H:4 * H]
    wh3 = wcat[din:, 0:3 * H]
    h0, c0, g0 = _k_d0(x, wfh)
    hn, gn, cn = _sc_gather_nbr([h0, g0, c0], bg_t)
    return _k_d1(x, hn, gn, cn, wh3)


def kernel(params, tree_fnode, tree_fmess, tree_agraph, tree_bgraph,
           tree_cgraph, roots, graph_fnode, graph_fmess, graph_agraph,
           graph_bgraph):
    p = params
    nt, mt = tree_fnode.shape[0], tree_fmess.shape[0]
    ng, mg = graph_fnode.shape[0], graph_fmess.shape[0]
    cl = tree_cgraph.shape[1]
    nroot = roots.shape[0]
    ntp, mtp = _ceil_to(nt, 2048), _ceil_to(mt, 2048)
    ngp, mgp = _ceil_to(ng, 2048), _ceil_to(mg, 2048)

    fnode_g_p = _pad_idx(graph_fnode, ngp)
    src_g = _pad_idx(graph_fmess[:, 0], mgp)
    bond_g = _pad_idx(graph_fmess[:, 2], mgp)
    pos_g = _pad_idx(graph_fmess[:, 3], mgp)
    bg_g = _pad_idx2(graph_bgraph.T, mgp)
    ag_g = _pad_idx2(graph_agraph.T, ngp)
    src_t = _pad_idx(tree_fmess[:, 0], mtp)
    pos_t = _pad_idx(tree_fmess[:, 2], mtp)
    bg_t = _pad_idx2(tree_bgraph.T, mtp)
    ag_t = _pad_idx2(tree_agraph.T, ntp)
    cg_t = _pad_idx2(tree_cgraph.T, ntp)
    tf0 = _pad_idx(tree_fnode[:, 0], ntp)
    tf1 = _pad_idx(tree_fnode[:, 1], ntp)
    tf2 = _pad_idx(tree_fnode[:, 2], ntp)

    wcat_g, bcat_g = _gate_weights(p['enc_graph'])
    atom_tab = wcat_g[0:ATOM_SIZE]
    bond_tab = _pad_rows(wcat_g[ATOM_SIZE:ATOM_SIZE + NBOND], 8)
    pos_tab_g = _pad_rows(wcat_g[ATOM_SIZE + NBOND:ATOM_SIZE + NBOND + MAX_POS]
                          + bcat_g[None, :], 24)
    atom_rows = _sc_gather(atom_tab, fnode_g_p)
    xg_g = _sc_gather(atom_rows, src_g)
    x_g = _k_xadd(xg_g, pos_g, pos_tab_g, bond_g, bond_tab)
    h1_g = _msg_pass(x_g, bg_g, wcat_g, ATOM_SIZE + NBOND + MAX_POS, mgp)
    wo_g, bo_g = p['enc_graph']['Wo']
    p_g = _sc_gather(wo_g[0:ATOM_SIZE], fnode_g_p)
    nn_g = _sc_gather_nbr([h1_g], ag_g)[0]
    hatom_p = _k_node(p_g, nn_g, wo_g[ATOM_SIZE:], bo_g, 4, True)

    wcat_b, bcat_b = _gate_weights(p['enc_bond'])
    hnode_b = _sc_gather(p['E_l'], tf2)
    t_b = _k_mm(hnode_b, wcat_b[0:H], bcat_b)
    x_b = _k_xadd(_sc_gather(t_b, src_t), pos_t,
                  _pad_rows(wcat_b[H:H + MAX_POS], 24))
    h1_b = _msg_pass(x_b, bg_t, wcat_b, H + MAX_POS, mtp)
    wo_b, bo_b = p['enc_bond']['Wo']
    p_b = _k_mm(hnode_b, wo_b[0:H], jnp.zeros((H,), F32))
    nn_b = _sc_gather_nbr([h1_b], ag_t)[0]
    hbond_p = _k_node(p_b, nn_b, wo_b[H:], bo_b, 4, True)

    w_i, b_i = p['W_i']
    ei_tab = _k_mm(_pad_rows(p['E_i'], _ceil_to(p['E_i'].shape[0], BM)),
                   w_i[0:H], jnp.zeros((H,), F32))
    eg_f = _sc_gather(ei_tab, tf1)
    cl_rows = _sc_gather_nbr([hatom_p], cg_t)[0]
    hnode_f = _k_node(eg_f, cl_rows, w_i[H:], b_i, cl, False)
    wcat_f, bcat_f = _gate_weights(p['enc_frag'])
    t_f = _k_mm(hnode_f, wcat_f[0:H], bcat_f)
    x_f = _k_xadd(_sc_gather(t_f, src_t), pos_t,
                  _pad_rows(wcat_f[H:H + MAX_POS], 24))
    h1_f = _msg_pass(x_f, bg_t, wcat_f, H + MAX_POS, mtp)
    wo_f, bo_f = p['enc_frag']['Wo']
    p_f = _k_mm(hnode_f, wo_f[0:H], jnp.zeros((H,), F32))
    nn_f = _sc_gather_nbr([h1_f], ag_t)[0]
    hinter_p = _k_node(p_f, nn_f, wo_f[H:], bo_f, 4, True)

    w_c, b_c = p['W_c']
    ec_tab = _k_mm(_pad_rows(p['E_c'], _ceil_to(p['E_c'].shape[0], BM)),
                   w_c[0:H], jnp.zeros((H,), F32))
    eg_c = _sc_gather(ec_tab, tf0)
    hnode_c = _k_hc(eg_c, hinter_p, hbond_p, w_c[H:2 * H], w_c[2 * H:], b_c)
    wcat_c, bcat_c = _gate_weights(p['enc_inter'])
    t_c = _k_mm(hnode_c, wcat_c[0:H], bcat_c)
    x_c = _k_xadd(_sc_gather(t_c, src_t), pos_t,
                  _pad_rows(wcat_c[H:H + MAX_POS], 24))
    h1_c = _msg_pass(x_c, bg_t, wcat_c, H + MAX_POS, mtp)
    wo_c, bo_c = p['enc_inter']['Wo']
    p_c = _k_mm(hnode_c, wo_c[0:H], jnp.zeros((H,), F32))
    nn_c = _sc_gather_nbr([h1_c], ag_t)[0]
    hnode_p = _k_node(p_c, nn_c, wo_c[H:], bo_c, 4, True)

    wr, br = p['W_root']
    roots_p = _pad_idx(roots, 256)
    fr = _sc_gather(hnode_c, roots_p)[:nroot]
    agr = _pad_idx(jnp.take(tree_agraph, roots, axis=0).reshape(-1), 256)
    rn = _sc_gather(h1_c, agr)[:nroot * 4].reshape(nroot, 4 * H)
    hroot = _k_root(fr, rn, wr[0:H], wr[H:], br)

    return (hroot, hnode_p[:nt], hinter_p[:nt], hbond_p[:nt], hatom_p[:ng])


# speedup vs baseline: 1.8567x; 1.2159x over previous
import functools

import jax
import jax.numpy as jnp
from jax import lax
from jax.experimental import pallas as pl
from jax.experimental.pallas import tpu as pltpu, tpu_sc as plsc

H = 128
MAX_POS = 20
ATOM_SIZE = 40
NBOND = 4
NW = 32
BM = 512
F32 = jnp.float32


def _pick_chunk(bpw, d):
    cap = max(8, min(bpw, (128 * 1024) // (d * 4)))
    for c in range(cap - cap % 8, 7, -8):
        if bpw % c == 0:
            return c
    return 8


def _pick_chunk2(bpw, d, a_dim, ntab):
    cap = max(8, min(bpw, 470_000 // (2 * ntab * a_dim * d * 4)))
    best = 8
    for c in range(8, cap + 1, 8):
        if bpw % c == 0:
            best = c
    return best


def _sc_gather_pipe(tables, idx_t, d):
    a_dim, mp = idx_t.shape
    ntab = len(tables)
    bpw = mp // NW
    chunk = _pick_chunk2(bpw, d, a_dim, ntab)
    nloop = bpw // chunk
    k_idx = a_dim * chunk
    idx_p = idx_t.reshape(a_dim, NW, nloop, chunk).transpose(1, 2, 0, 3).reshape(-1)
    mesh = plsc.VectorSubcoreMesh(core_axis_name="c", subcore_axis_name="s")
    shp = jax.ShapeDtypeStruct((a_dim * mp, d), F32)

    @functools.partial(
        pl.kernel,
        out_type=tuple(shp for _ in range(ntab)),
        mesh=mesh,
        scratch_types=[
            pltpu.VMEM((k_idx,), jnp.int32),
            pltpu.VMEM((k_idx,), jnp.int32),
            pltpu.VMEM((ntab * k_idx, d), F32),
            pltpu.VMEM((ntab * k_idx, d), F32),
            pltpu.SemaphoreType.DMA,
            pltpu.SemaphoreType.DMA,
            pltpu.SemaphoreType.DMA,
            pltpu.SemaphoreType.DMA,
            pltpu.SemaphoreType.DMA,
        ],
    )
    def k(*refs):
        tabs = refs[:ntab]
        idx_hbm = refs[ntab]
        outs = refs[ntab + 1:ntab + 1 + ntab]
        (i0, i1, r0, r1, is0, is1, gsem, ss0, ss1) = refs[ntab + 1 + ntab:]
        idxb, rowsb, isems, ssems = [i0, i1], [r0, r1], [is0, is1], [ss0, ss1]
        wid = lax.axis_index("s") * 2 + lax.axis_index("c")
        ibase = wid * nloop * k_idx
        obase = wid * bpw

        def idx_src(j):
            return idx_hbm.at[pl.ds(ibase + j * k_idx, k_idx)]

        def out_reg(t, a, j):
            return outs[t].at[pl.ds(a * mp + obase + j * chunk, chunk)]

        def rslice(rb, t, a):
            return rb.at[pl.ds((t * a_dim + a) * chunk, chunk)]

        def _when(cond, fn):
            if isinstance(cond, bool):
                if cond:
                    fn()
            else:
                pl.when(cond)(fn)

        def step(j, b):
            def drain_prev():
                for t in range(ntab):
                    for a in range(a_dim):
                        pltpu.make_async_copy(
                            rslice(rowsb[b], t, a), out_reg(t, a, j - 2),
                            ssems[b]).wait()
            _when(j >= 2, drain_prev)
            pltpu.make_async_copy(idx_src(j), idxb[b], isems[b]).wait()
            descs = [
                pltpu.async_copy(
                    tabs[t].at[idxb[b]],
                    rowsb[b].at[pl.ds(t * k_idx, k_idx)], gsem)
                for t in range(ntab)
            ]
            for de in descs:
                de.wait()

            def prefetch():
                pltpu.async_copy(idx_src(j + 2), idxb[b], isems[b])
            _when(j + 2 < nloop, prefetch)
            for t in range(ntab):
                for a in range(a_dim):
                    pltpu.async_copy(rslice(rowsb[b], t, a), out_reg(t, a, j),
                                     ssems[b])

        pltpu.async_copy(idx_src(0), idxb[0], isems[0])
        if nloop > 1:
            pltpu.async_copy(idx_src(1), idxb[1], isems[1])

        def body2(j0, carry):
            step(j0 * 2, 0)
            step(j0 * 2 + 1, 1)
            return carry

        lax.fori_loop(0, nloop // 2, body2, 0)
        if nloop % 2:
            step(nloop - 1, (nloop - 1) % 2)
        for jl in range(max(0, nloop - 2), nloop):
            b = jl % 2
            for t in range(ntab):
                for a in range(a_dim):
                    pltpu.make_async_copy(
                        rslice(rowsb[b], t, a), out_reg(t, a, jl),
                        ssems[b]).wait()

    out = k(*tables, idx_p)
    return tuple(out) if isinstance(out, (list, tuple)) else (out,)


def _sc_gather(table, idx):
    return _sc_gather_pipe([table], idx[None, :], table.shape[1])[0]


def _sc_gather_nbr(tables, idx_t):
    a_dim, mp = idx_t.shape
    outs = _sc_gather_pipe(tables, idx_t, H)
    return tuple(o.reshape(a_dim, mp, H) for o in outs)


def _row(d):
    return pl.BlockSpec((BM, d), lambda i: (i, 0))


def _full(shape):
    return pl.BlockSpec(shape, lambda i: tuple(0 for _ in shape))


def _rowmask(x):
    rows = pl.program_id(0) * BM + lax.broadcasted_iota(jnp.int32, (BM, 1), 0)
    return x * (rows != 0).astype(F32)


def _dot(a, b):
    return jnp.dot(a, b, preferred_element_type=F32)


def _sigm(x):
    return jax.nn.sigmoid(x)


def _k_mm(x, w, b, relu=False):
    m, kdim = x.shape
    n = w.shape[1]

    def body(x_ref, w_ref, b_ref, o_ref):
        acc = _dot(x_ref[...], w_ref[...]) + b_ref[...]
        o_ref[...] = jnp.maximum(acc, 0.0) if relu else acc

    return pl.pallas_call(
        body,
        grid=(m // BM,),
        in_specs=[_row(kdim), _full((kdim, n)), _full((1, n))],
        out_specs=_row(n),
        out_shape=jax.ShapeDtypeStruct((m, n), F32),
    )(x, w, b.reshape(1, n))


def _k_xadd(xg, pos, ptab, bond=None, btab=None):
    m = xg.shape[0]
    n = xg.shape[1]
    pw = ptab.shape[0]

    def body(*refs):
        if btab is None:
            xg_ref, pos_ref, ptab_ref, o_ref = refs
        else:
            xg_ref, pos_ref, bond_ref, ptab_ref, btab_ref, o_ref = refs
        acc = xg_ref[...]
        oh_p = (pos_ref[...] == lax.broadcasted_iota(jnp.int32, (BM, pw), 1)
                ).astype(F32)
        acc = acc + _dot(oh_p, ptab_ref[...])
        if btab is not None:
            bw = btab.shape[0]
            oh_b = (bond_ref[...] == lax.broadcasted_iota(jnp.int32, (BM, bw), 1)
                    ).astype(F32)
            acc = acc + _dot(oh_b, btab_ref[...])
        o_ref[...] = acc

    specs = [_row(n), pl.BlockSpec((BM, 1), lambda i: (i, 0))]
    args = [xg, pos.reshape(m, 1)]
    if btab is not None:
        specs.append(pl.BlockSpec((BM, 1), lambda i: (i, 0)))
        args.append(bond.reshape(m, 1))
    specs.append(_full(ptab.shape))
    args.append(ptab)
    if btab is not None:
        specs.append(_full(btab.shape))
        args.append(btab)

    return pl.pallas_call(
        body,
        grid=(m // BM,),
        in_specs=specs,
        out_specs=_row(n),
        out_shape=jax.ShapeDtypeStruct((m, n), F32),
    )(*args)


def _k_d0(x, wfh):
    m = x.shape[0]

    def body(x_ref, wfh_ref, h_ref, c_ref, g_ref):
        xv = x_ref[...]
        i = _sigm(xv[:, 0:H])
        o = _sigm(xv[:, H:2 * H])
        u = jnp.tanh(xv[:, 2 * H:3 * H])
        c = _rowmask(i * u)
        h = _rowmask(o * jnp.tanh(c))
        h_ref[...] = h
        c_ref[...] = c
        g_ref[...] = _dot(h, wfh_ref[...])

    shp = jax.ShapeDtypeStruct((m, H), F32)
    return pl.pallas_call(
        body,
        grid=(m // BM,),
        in_specs=[_row(4 * H), _full((H, H))],
        out_specs=(_row(H), _row(H), _row(H)),
        out_shape=(shp, shp, shp),
    )(x, wfh)


def _k_d1(x, hn, gn, cn, wh3):
    m = x.shape[0]

    nbr = pl.BlockSpec((4, BM, H), lambda i: (0, i, 0))

    def body(x_ref, hn_ref, gn_ref, cn_ref, wh3_ref, h_ref):
        xv = x_ref[...]
        hs = (hn_ref[0] + hn_ref[1] + hn_ref[2] + hn_ref[3])
        z = _dot(hs, wh3_ref[...])
        i = _sigm(xv[:, 0:H] + z[:, 0:H])
        o = _sigm(xv[:, H:2 * H] + z[:, H:2 * H])
        u = jnp.tanh(xv[:, 2 * H:3 * H] + z[:, 2 * H:3 * H])
        xf = xv[:, 3 * H:4 * H]
        fc = jnp.zeros((BM, H), F32)
        for a in range(4):
            fc = fc + _sigm(xf + gn_ref[a]) * cn_ref[a]
        c = i * u + fc
        h_ref[...] = _rowmask(o * jnp.tanh(c))

    return pl.pallas_call(
        body,
        grid=(m // BM,),
        in_specs=[_row(4 * H), nbr, nbr, nbr, _full((H, 3 * H))],
        out_specs=_row(H),
        out_shape=jax.ShapeDtypeStruct((m, H), F32),
    )(x, hn, gn, cn, wh3)


def _k_node(p, nn, w, b, na, mask0):
    m = p.shape[0]

    def body(p_ref, nn_ref, w_ref, b_ref, o_ref):
        hs = jnp.zeros((BM, H), F32)
        for a in range(na):
            hs = hs + nn_ref[a]
        out = jnp.maximum(p_ref[...] + _dot(hs, w_ref[...]) + b_ref[...], 0.0)
        o_ref[...] = _rowmask(out) if mask0 else out

    return pl.pallas_call(
        body,
        grid=(m // BM,),
        in_specs=[_row(H), pl.BlockSpec((na, BM, H), lambda i: (0, i, 0)),
                  _full((H, H)), _full((1, H))],
        out_specs=_row(H),
        out_shape=jax.ShapeDtypeStruct((m, H), F32),
    )(p, nn, w, b.reshape(1, H))


def _k_hc(eg, hi, hb, w1, w2, b):
    m = eg.shape[0]

    def body(eg_ref, hi_ref, hb_ref, w1_ref, w2_ref, b_ref, o_ref):
        o_ref[...] = jnp.maximum(
            eg_ref[...] + _dot(hi_ref[...], w1_ref[...])
            + _dot(hb_ref[...], w2_ref[...]) + b_ref[...], 0.0)

    return pl.pallas_call(
        body,
        grid=(m // BM,),
        in_specs=[_row(H), _row(H), _row(H), _full((H, H)), _full((H, H)),
                  _full((1, H))],
        out_specs=_row(H),
        out_shape=jax.ShapeDtypeStruct((m, H), F32),
    )(eg, hi, hb, w1, w2, b.reshape(1, H))


def _k_root(fr, rn, w1, w2, b):
    nr = fr.shape[0]

    def body(fr_ref, rn_ref, w1_ref, w2_ref, b_ref, o_ref):
        rnv = rn_ref[...]
        hs = (rnv[:, 0:H] + rnv[:, H:2 * H]
              + rnv[:, 2 * H:3 * H] + rnv[:, 3 * H:4 * H])
        o_ref[...] = jnp.tanh(_dot(fr_ref[...], w1_ref[...])
                              + _dot(hs, w2_ref[...]) + b_ref[...])

    return pl.pallas_call(
        body,
        grid=(1,),
        in_specs=[_full((nr, H)), _full((nr, 4 * H)), _full((H, H)),
                  _full((H, H)), _full((1, H))],
        out_specs=_full((nr, H)),
        out_shape=jax.ShapeDtypeStruct((nr, H), F32),
    )(fr, rn, w1, w2, b.reshape(1, H))


def _ceil_to(x, m):
    return (x + m - 1) // m * m


def _pad_rows(x, n):
    return jnp.pad(x, ((0, n - x.shape[0]),) + ((0, 0),) * (x.ndim - 1))


def _pad_idx(idx, n):
    return jnp.pad(idx.astype(jnp.int32), (0, n - idx.shape[0]))


def _gate_weights(p):
    wcat = jnp.concatenate(
        [p['Wi'][0], p['Wog'][0], p['Wu'][0], p['Wf'][0]], axis=1)
    bcat = jnp.concatenate(
        [p['Wi'][1], p['Wog'][1], p['Wu'][1], p['Wf'][1]])
    return wcat, bcat


def _pad_idx2(idx, n):
    return jnp.pad(idx.astype(jnp.int32), ((0, 0), (0, n - idx.shape[1])))


def _msg_pass(x, bg_t, wcat, din, mp):
    wfh = wcat[din:, 3 * H:4 * H]
    wh3 = wcat[din:, 0:3 * H]
    h0, c0, g0 = _k_d0(x, wfh)
    hn, gn, cn = _sc_gather_nbr([h0, g0, c0], bg_t)
    return _k_d1(x, hn, gn, cn, wh3)


def kernel(params, tree_fnode, tree_fmess, tree_agraph, tree_bgraph,
           tree_cgraph, roots, graph_fnode, graph_fmess, graph_agraph,
           graph_bgraph):
    p = params
    nt, mt = tree_fnode.shape[0], tree_fmess.shape[0]
    ng, mg = graph_fnode.shape[0], graph_fmess.shape[0]
    cl = tree_cgraph.shape[1]
    nroot = roots.shape[0]
    ntp, mtp = _ceil_to(nt, 2048), _ceil_to(mt, 2048)
    ngp, mgp = _ceil_to(ng, 2048), _ceil_to(mg, 2048)

    fnode_g_p = _pad_idx(graph_fnode, ngp)
    src_g = _pad_idx(graph_fmess[:, 0], mgp)
    bond_g = _pad_idx(graph_fmess[:, 2], mgp)
    pos_g = _pad_idx(graph_fmess[:, 3], mgp)
    bg_g = _pad_idx2(graph_bgraph.T, mgp)
    ag_g = _pad_idx2(graph_agraph.T, ngp)
    src_t = _pad_idx(tree_fmess[:, 0], mtp)
    pos_t = _pad_idx(tree_fmess[:, 2], mtp)
    bg_t = _pad_idx2(tree_bgraph.T, mtp)
    ag_t = _pad_idx2(tree_agraph.T, ntp)
    cg_t = _pad_idx2(tree_cgraph.T, ntp)
    tf0 = _pad_idx(tree_fnode[:, 0], ntp)
    tf1 = _pad_idx(tree_fnode[:, 1], ntp)
    tf2 = _pad_idx(tree_fnode[:, 2], ntp)

    wcat_g, bcat_g = _gate_weights(p['enc_graph'])
    atom_tab = wcat_g[0:ATOM_SIZE]
    bond_tab = _pad_rows(wcat_g[ATOM_SIZE:ATOM_SIZE + NBOND], 8)
    pos_tab_g = _pad_rows(wcat_g[ATOM_SIZE + NBOND:ATOM_SIZE + NBOND + MAX_POS]
                          + bcat_g[None, :], 24)
    atom_rows = _sc_gather(atom_tab, fnode_g_p)
    xg_g = _sc_gather(atom_rows, src_g)
    x_g = _k_xadd(xg_g, pos_g, pos_tab_g, bond_g, bond_tab)
    h1_g = _msg_pass(x_g, bg_g, wcat_g, ATOM_SIZE + NBOND + MAX_POS, mgp)
    wo_g, bo_g = p['enc_graph']['Wo']
    p_g = _sc_gather(wo_g[0:ATOM_SIZE], fnode_g_p)
    nn_g = _sc_gather_nbr([h1_g], ag_g)[0]
    hatom_p = _k_node(p_g, nn_g, wo_g[ATOM_SIZE:], bo_g, 4, True)

    wcat_b, bcat_b = _gate_weights(p['enc_bond'])
    hnode_b = _sc_gather(p['E_l'], tf2)
    t_b = _k_mm(hnode_b, wcat_b[0:H], bcat_b)
    x_b = _k_xadd(_sc_gather(t_b, src_t), pos_t,
                  _pad_rows(wcat_b[H:H + MAX_POS], 24))
    h1_b = _msg_pass(x_b, bg_t, wcat_b, H + MAX_POS, mtp)
    wo_b, bo_b = p['enc_bond']['Wo']
    p_b = _k_mm(hnode_b, wo_b[0:H], jnp.zeros((H,), F32))
    nn_b = _sc_gather_nbr([h1_b], ag_t)[0]
    hbond_p = _k_node(p_b, nn_b, wo_b[H:], bo_b, 4, True)

    w_i, b_i = p['W_i']
    ei_tab = _k_mm(_pad_rows(p['E_i'], _ceil_to(p['E_i'].shape[0], BM)),
                   w_i[0:H], jnp.zeros((H,), F32))
    eg_f = _sc_gather(ei_tab, tf1)
    cl_rows = _sc_gather_nbr([hatom_p], cg_t)[0]
    hnode_f = _k_node(eg_f, cl_rows, w_i[H:], b_i, cl, False)
    wcat_f, bcat_f = _gate_weights(p['enc_frag'])
    t_f = _k_mm(hnode_f, wcat_f[0:H], bcat_f)
    x_f = _k_xadd(_sc_gather(t_f, src_t), pos_t,
                  _pad_rows(wcat_f[H:H + MAX_POS], 24))
    h1_f = _msg_pass(x_f, bg_t, wcat_f, H + MAX_POS, mtp)
    wo_f, bo_f = p['enc_frag']['Wo']
    p_f = _k_mm(hnode_f, wo_f[0:H], jnp.zeros((H,), F32))
    nn_f = _sc_gather_nbr([h1_f], ag_t)[0]
    hinter_p = _k_node(p_f, nn_f, wo_f[H:], bo_f, 4, True)

    w_c, b_c = p['W_c']
    ec_tab = _k_mm(_pad_rows(p['E_c'], _ceil_to(p['E_c'].shape[0], BM)),
                   w_c[0:H], jnp.zeros((H,), F32))
    eg_c = _sc_gather(ec_tab, tf0)
    hnode_c = _k_hc(eg_c, hinter_p, hbond_p, w_c[H:2 * H], w_c[2 * H:], b_c)
    wcat_c, bcat_c = _gate_weights(p['enc_inter'])
    t_c = _k_mm(hnode_c, wcat_c[0:H], bcat_c)
    x_c = _k_xadd(_sc_gather(t_c, src_t), pos_t,
                  _pad_rows(wcat_c[H:H + MAX_POS], 24))
    h1_c = _msg_pass(x_c, bg_t, wcat_c, H + MAX_POS, mtp)
    wo_c, bo_c = p['enc_inter']['Wo']
    p_c = _k_mm(hnode_c, wo_c[0:H], jnp.zeros((H,), F32))
    nn_c = _sc_gather_nbr([h1_c], ag_t)[0]
    hnode_p = _k_node(p_c, nn_c, wo_c[H:], bo_c, 4, True)

    wr, br = p['W_root']
    roots_p = _pad_idx(roots, 256)
    fr = _sc_gather(hnode_c, roots_p)[:nroot]
    agr = _pad_idx(jnp.take(tree_agraph, roots, axis=0).reshape(-1), 256)
    rn = _sc_gather(h1_c, agr)[:nroot * 4].reshape(nroot, 4 * H)
    hroot = _k_root(fr, rn, wr[0:H], wr[H:], br)

    return (hroot, hnode_p[:nt], hinter_p[:nt], hbond_p[:nt], hatom_p[:ng])


# speedup vs baseline: 1.9791x; 1.0659x over previous
import functools

import jax
import jax.numpy as jnp
from jax import lax
from jax.experimental import pallas as pl
from jax.experimental.pallas import tpu as pltpu, tpu_sc as plsc

H = 128
MAX_POS = 20
ATOM_SIZE = 40
NBOND = 4
NW = 32
BM = 512
F32 = jnp.float32


def _pick_chunk(bpw, d):
    cap = max(8, min(bpw, (128 * 1024) // (d * 4)))
    for c in range(cap - cap % 8, 7, -8):
        if bpw % c == 0:
            return c
    return 8


def _pick_chunk2(bpw, d, a_dim, ntab):
    cap = max(8, min(bpw, 470_000 // (2 * ntab * a_dim * d * 4)))
    best = 8
    for c in range(8, cap + 1, 8):
        if bpw % c == 0:
            best = c
    return best


def _sc_gather_pipe(tables, idx_t, d):
    a_dim, mp = idx_t.shape
    ntab = len(tables)
    bpw = mp // NW
    chunk = _pick_chunk2(bpw, d, a_dim, ntab)
    nloop = bpw // chunk
    k_idx = a_dim * chunk
    idx_p = idx_t.reshape(a_dim, NW, nloop, chunk).transpose(1, 2, 0, 3).reshape(-1)
    mesh = plsc.VectorSubcoreMesh(core_axis_name="c", subcore_axis_name="s")
    shp = jax.ShapeDtypeStruct((a_dim * mp, d), F32)

    @functools.partial(
        pl.kernel,
        out_type=tuple(shp for _ in range(ntab)),
        mesh=mesh,
        scratch_types=[
            pltpu.VMEM((k_idx,), jnp.int32),
            pltpu.VMEM((k_idx,), jnp.int32),
            pltpu.VMEM((ntab * k_idx, d), F32),
            pltpu.VMEM((ntab * k_idx, d), F32),
            pltpu.SemaphoreType.DMA,
            pltpu.SemaphoreType.DMA,
            pltpu.SemaphoreType.DMA,
            pltpu.SemaphoreType.DMA,
            pltpu.SemaphoreType.DMA,
        ],
    )
    def k(*refs):
        tabs = refs[:ntab]
        idx_hbm = refs[ntab]
        outs = refs[ntab + 1:ntab + 1 + ntab]
        (i0, i1, r0, r1, is0, is1, gsem, ss0, ss1) = refs[ntab + 1 + ntab:]
        idxb, rowsb, isems, ssems = [i0, i1], [r0, r1], [is0, is1], [ss0, ss1]
        wid = lax.axis_index("s") * 2 + lax.axis_index("c")
        ibase = wid * nloop * k_idx
        obase = wid * bpw

        def idx_src(j):
            return idx_hbm.at[pl.ds(ibase + j * k_idx, k_idx)]

        def out_reg(t, a, j):
            return outs[t].at[pl.ds(a * mp + obase + j * chunk, chunk)]

        def rslice(rb, t, a):
            return rb.at[pl.ds((t * a_dim + a) * chunk, chunk)]

        def _when(cond, fn):
            if isinstance(cond, bool):
                if cond:
                    fn()
            else:
                pl.when(cond)(fn)

        def step(j, b):
            def drain_prev():
                for t in range(ntab):
                    for a in range(a_dim):
                        pltpu.make_async_copy(
                            rslice(rowsb[b], t, a), out_reg(t, a, j - 2),
                            ssems[b]).wait()
            _when(j >= 2, drain_prev)
            pltpu.make_async_copy(idx_src(j), idxb[b], isems[b]).wait()
            descs = [
                pltpu.async_copy(
                    tabs[t].at[idxb[b]],
                    rowsb[b].at[pl.ds(t * k_idx, k_idx)], gsem)
                for t in range(ntab)
            ]
            for de in descs:
                de.wait()

            def prefetch():
                pltpu.async_copy(idx_src(j + 2), idxb[b], isems[b])
            _when(j + 2 < nloop, prefetch)
            for t in range(ntab):
                for a in range(a_dim):
                    pltpu.async_copy(rslice(rowsb[b], t, a), out_reg(t, a, j),
                                     ssems[b])

        pltpu.async_copy(idx_src(0), idxb[0], isems[0])
        if nloop > 1:
            pltpu.async_copy(idx_src(1), idxb[1], isems[1])

        def body2(j0, carry):
            step(j0 * 2, 0)
            step(j0 * 2 + 1, 1)
            return carry

        lax.fori_loop(0, nloop // 2, body2, 0)
        if nloop % 2:
            step(nloop - 1, (nloop - 1) % 2)
        for jl in range(max(0, nloop - 2), nloop):
            b = jl % 2
            for t in range(ntab):
                for a in range(a_dim):
                    pltpu.make_async_copy(
                        rslice(rowsb[b], t, a), out_reg(t, a, jl),
                        ssems[b]).wait()

    out = k(*tables, idx_p)
    return tuple(out) if isinstance(out, (list, tuple)) else (out,)


def _sc_gather(table, idx):
    return _sc_gather_pipe([table], idx[None, :], table.shape[1])[0]


def _sc_gather_nbr(tables, idx_t):
    a_dim, mp = idx_t.shape
    outs = _sc_gather_pipe(tables, idx_t, H)
    return tuple(o.reshape(a_dim, mp, H) for o in outs)


def _row(d):
    return pl.BlockSpec((BM, d), lambda i: (i, 0))


def _full(shape):
    return pl.BlockSpec(shape, lambda i: tuple(0 for _ in shape))


def _rowmask(x):
    rows = pl.program_id(0) * BM + lax.broadcasted_iota(jnp.int32, (BM, 1), 0)
    return x * (rows != 0).astype(F32)


def _dot(a, b):
    return jnp.dot(a, b, preferred_element_type=F32)


def _sigm(x):
    return jax.nn.sigmoid(x)


def _k_mm(x, w, b, relu=False):
    m, kdim = x.shape
    n = w.shape[1]

    def body(x_ref, w_ref, b_ref, o_ref):
        acc = _dot(x_ref[...], w_ref[...]) + b_ref[...]
        o_ref[...] = jnp.maximum(acc, 0.0) if relu else acc

    return pl.pallas_call(
        body,
        grid=(m // BM,),
        in_specs=[_row(kdim), _full((kdim, n)), _full((1, n))],
        out_specs=_row(n),
        out_shape=jax.ShapeDtypeStruct((m, n), F32),
    )(x, w, b.reshape(1, n))


def _xparts(xg, pos_ref, ptab_ref, bond_ref, btab_ref, pw, bw):
    oh_p = (pos_ref[...] == lax.broadcasted_iota(jnp.int32, (BM, pw), 1)
            ).astype(F32)
    x = xg + _dot(oh_p, ptab_ref[...])
    if btab_ref is not None:
        oh_b = (bond_ref[...] == lax.broadcasted_iota(jnp.int32, (BM, bw), 1)
                ).astype(F32)
        x = x + _dot(oh_b, btab_ref[...])
    return x


def _k_d0(xg, pos, ptab, wfh, bond=None, btab=None):
    m = xg.shape[0]
    pw = ptab.shape[0]
    bw = btab.shape[0] if btab is not None else 0

    def body(*refs):
        if btab is None:
            xg_ref, pos_ref, ptab_ref, wfh_ref, h_ref, c_ref, g_ref = refs
            bond_ref = btab_ref = None
        else:
            (xg_ref, pos_ref, bond_ref, ptab_ref, btab_ref, wfh_ref,
             h_ref, c_ref, g_ref) = refs
        xv = _xparts(xg_ref[...], pos_ref, ptab_ref, bond_ref, btab_ref, pw, bw)
        i = _sigm(xv[:, 0:H])
        o = _sigm(xv[:, H:2 * H])
        u = jnp.tanh(xv[:, 2 * H:3 * H])
        c = _rowmask(i * u)
        h = _rowmask(o * jnp.tanh(c))
        h_ref[...] = h
        c_ref[...] = c
        g_ref[...] = _dot(h, wfh_ref[...])

    specs = [_row(4 * H), pl.BlockSpec((BM, 1), lambda i: (i, 0))]
    args = [xg, pos.reshape(m, 1)]
    if btab is not None:
        specs.append(pl.BlockSpec((BM, 1), lambda i: (i, 0)))
        args.append(bond.reshape(m, 1))
    specs.append(_full(ptab.shape))
    args.append(ptab)
    if btab is not None:
        specs.append(_full(btab.shape))
        args.append(btab)
    specs.append(_full((H, H)))
    args.append(wfh)

    shp = jax.ShapeDtypeStruct((m, H), F32)
    return pl.pallas_call(
        body,
        grid=(m // BM,),
        in_specs=specs,
        out_specs=(_row(H), _row(H), _row(H)),
        out_shape=(shp, shp, shp),
    )(*args)


def _k_d1(xg, pos, ptab, hn, gn, cn, wh3, bond=None, btab=None):
    m = xg.shape[0]
    pw = ptab.shape[0]
    bw = btab.shape[0] if btab is not None else 0
    nbr = pl.BlockSpec((4, BM, H), lambda i: (0, i, 0))

    def body(*refs):
        if btab is None:
            (xg_ref, pos_ref, ptab_ref, hn_ref, gn_ref, cn_ref, wh3_ref,
             h_ref) = refs
            bond_ref = btab_ref = None
        else:
            (xg_ref, pos_ref, bond_ref, ptab_ref, btab_ref, hn_ref, gn_ref,
             cn_ref, wh3_ref, h_ref) = refs
        xv = _xparts(xg_ref[...], pos_ref, ptab_ref, bond_ref, btab_ref, pw, bw)
        hs = (hn_ref[0] + hn_ref[1] + hn_ref[2] + hn_ref[3])
        z = _dot(hs, wh3_ref[...])
        i = _sigm(xv[:, 0:H] + z[:, 0:H])
        o = _sigm(xv[:, H:2 * H] + z[:, H:2 * H])
        u = jnp.tanh(xv[:, 2 * H:3 * H] + z[:, 2 * H:3 * H])
        xf = xv[:, 3 * H:4 * H]
        fc = jnp.zeros((BM, H), F32)
        for a in range(4):
            fc = fc + _sigm(xf + gn_ref[a]) * cn_ref[a]
        c = i * u + fc
        h_ref[...] = _rowmask(o * jnp.tanh(c))

    specs = [_row(4 * H), pl.BlockSpec((BM, 1), lambda i: (i, 0))]
    args = [xg, pos.reshape(m, 1)]
    if btab is not None:
        specs.append(pl.BlockSpec((BM, 1), lambda i: (i, 0)))
        args.append(bond.reshape(m, 1))
    specs.append(_full(ptab.shape))
    args.append(ptab)
    if btab is not None:
        specs.append(_full(btab.shape))
        args.append(btab)
    specs += [nbr, nbr, nbr, _full((H, 3 * H))]
    args += [hn, gn, cn, wh3]

    return pl.pallas_call(
        body,
        grid=(m // BM,),
        in_specs=specs,
        out_specs=_row(H),
        out_shape=jax.ShapeDtypeStruct((m, H), F32),
    )(*args)


def _k_node(p, nn, w, b, na, mask0):
    m = p.shape[0]

    def body(p_ref, nn_ref, w_ref, b_ref, o_ref):
        hs = jnp.zeros((BM, H), F32)
        for a in range(na):
            hs = hs + nn_ref[a]
        out = jnp.maximum(p_ref[...] + _dot(hs, w_ref[...]) + b_ref[...], 0.0)
        o_ref[...] = _rowmask(out) if mask0 else out

    return pl.pallas_call(
        body,
        grid=(m // BM,),
        in_specs=[_row(H), pl.BlockSpec((na, BM, H), lambda i: (0, i, 0)),
                  _full((H, H)), _full((1, H))],
        out_specs=_row(H),
        out_shape=jax.ShapeDtypeStruct((m, H), F32),
    )(p, nn, w, b.reshape(1, H))


def _k_hc(eg, hi, hb, w1, w2, b):
    m = eg.shape[0]

    def body(eg_ref, hi_ref, hb_ref, w1_ref, w2_ref, b_ref, o_ref):
        o_ref[...] = jnp.maximum(
            eg_ref[...] + _dot(hi_ref[...], w1_ref[...])
            + _dot(hb_ref[...], w2_ref[...]) + b_ref[...], 0.0)

    return pl.pallas_call(
        body,
        grid=(m // BM,),
        in_specs=[_row(H), _row(H), _row(H), _full((H, H)), _full((H, H)),
                  _full((1, H))],
        out_specs=_row(H),
        out_shape=jax.ShapeDtypeStruct((m, H), F32),
    )(eg, hi, hb, w1, w2, b.reshape(1, H))


def _k_root(fr, rn, w1, w2, b):
    nr = fr.shape[0]

    def body(fr_ref, rn_ref, w1_ref, w2_ref, b_ref, o_ref):
        rnv = rn_ref[...]
        hs = (rnv[:, 0:H] + rnv[:, H:2 * H]
              + rnv[:, 2 * H:3 * H] + rnv[:, 3 * H:4 * H])
        o_ref[...] = jnp.tanh(_dot(fr_ref[...], w1_ref[...])
                              + _dot(hs, w2_ref[...]) + b_ref[...])

    return pl.pallas_call(
        body,
        grid=(1,),
        in_specs=[_full((nr, H)), _full((nr, 4 * H)), _full((H, H)),
                  _full((H, H)), _full((1, H))],
        out_specs=_full((nr, H)),
        out_shape=jax.ShapeDtypeStruct((nr, H), F32),
    )(fr, rn, w1, w2, b.reshape(1, H))


def _ceil_to(x, m):
    return (x + m - 1) // m * m


def _pad_rows(x, n):
    return jnp.pad(x, ((0, n - x.shape[0]),) + ((0, 0),) * (x.ndim - 1))


def _pad_idx(idx, n):
    return jnp.pad(idx.astype(jnp.int32), (0, n - idx.shape[0]))


def _gate_weights(p):
    wcat = jnp.concatenate(
        [p['Wi'][0], p['Wog'][0], p['Wu'][0], p['Wf'][0]], axis=1)
    bcat = jnp.concatenate(
        [p['Wi'][1], p['Wog'][1], p['Wu'][1], p['Wf'][1]])
    return wcat, bcat


def _pad_idx2(idx, n):
    return jnp.pad(idx.astype(jnp.int32), ((0, 0), (0, n - idx.shape[1])))


def _msg_pass(xg, pos, ptab, bg_t, wcat, din, mp, bond=None, btab=None):
    wfh = wcat[din:, 3 * H:4 * H]
    wh3 = wcat[din:, 0:3 * H]
    h0, c0, g0 = _k_d0(xg, pos, ptab, wfh, bond, btab)
    hn, gn, cn = _sc_gather_nbr([h0, g0, c0], bg_t)
    return _k_d1(xg, pos, ptab, hn, gn, cn, wh3, bond, btab)


def kernel(params, tree_fnode, tree_fmess, tree_agraph, tree_bgraph,
           tree_cgraph, roots, graph_fnode, graph_fmess, graph_agraph,
           graph_bgraph):
    p = params
    nt, mt = tree_fnode.shape[0], tree_fmess.shape[0]
    ng, mg = graph_fnode.shape[0], graph_fmess.shape[0]
    cl = tree_cgraph.shape[1]
    nroot = roots.shape[0]
    ntp, mtp = _ceil_to(nt, 2048), _ceil_to(mt, 2048)
    ngp, mgp = _ceil_to(ng, 2048), _ceil_to(mg, 2048)

    fnode_g_p = _pad_idx(graph_fnode, ngp)
    src_g = _pad_idx(graph_fmess[:, 0], mgp)
    bond_g = _pad_idx(graph_fmess[:, 2], mgp)
    pos_g = _pad_idx(graph_fmess[:, 3], mgp)
    bg_g = _pad_idx2(graph_bgraph.T, mgp)
    ag_g = _pad_idx2(graph_agraph.T, ngp)
    src_t = _pad_idx(tree_fmess[:, 0], mtp)
    pos_t = _pad_idx(tree_fmess[:, 2], mtp)
    bg_t = _pad_idx2(tree_bgraph.T, mtp)
    ag_t = _pad_idx2(tree_agraph.T, ntp)
    cg_t = _pad_idx2(tree_cgraph.T, ntp)
    tf0 = _pad_idx(tree_fnode[:, 0], ntp)
    tf1 = _pad_idx(tree_fnode[:, 1], ntp)
    tf2 = _pad_idx(tree_fnode[:, 2], ntp)

    wcat_g, bcat_g = _gate_weights(p['enc_graph'])
    atom_tab = wcat_g[0:ATOM_SIZE]
    bond_tab = _pad_rows(wcat_g[ATOM_SIZE:ATOM_SIZE + NBOND], 8)
    pos_tab_g = _pad_rows(wcat_g[ATOM_SIZE + NBOND:ATOM_SIZE + NBOND + MAX_POS]
                          + bcat_g[None, :], 24)
    atom_rows = _sc_gather(atom_tab, fnode_g_p)
    xg_g = _sc_gather(atom_rows, src_g)
    h1_g = _msg_pass(xg_g, pos_g, pos_tab_g, bg_g, wcat_g,
                     ATOM_SIZE + NBOND + MAX_POS, mgp, bond_g, bond_tab)
    wo_g, bo_g = p['enc_graph']['Wo']
    p_g = _sc_gather(wo_g[0:ATOM_SIZE], fnode_g_p)
    nn_g = _sc_gather_nbr([h1_g], ag_g)[0]
    hatom_p = _k_node(p_g, nn_g, wo_g[ATOM_SIZE:], bo_g, 4, True)

    wcat_b, bcat_b = _gate_weights(p['enc_bond'])
    hnode_b = _sc_gather(p['E_l'], tf2)
    t_b = _k_mm(hnode_b, wcat_b[0:H], bcat_b)
    h1_b = _msg_pass(_sc_gather(t_b, src_t), pos_t,
                     _pad_rows(wcat_b[H:H + MAX_POS], 24),
                     bg_t, wcat_b, H + MAX_POS, mtp)
    wo_b, bo_b = p['enc_bond']['Wo']
    p_b = _k_mm(hnode_b, wo_b[0:H], jnp.zeros((H,), F32))
    nn_b = _sc_gather_nbr([h1_b], ag_t)[0]
    hbond_p = _k_node(p_b, nn_b, wo_b[H:], bo_b, 4, True)

    w_i, b_i = p['W_i']
    ei_tab = _k_mm(_pad_rows(p['E_i'], _ceil_to(p['E_i'].shape[0], BM)),
                   w_i[0:H], jnp.zeros((H,), F32))
    eg_f = _sc_gather(ei_tab, tf1)
    cl_rows = _sc_gather_nbr([hatom_p], cg_t)[0]
    hnode_f = _k_node(eg_f, cl_rows, w_i[H:], b_i, cl, False)
    wcat_f, bcat_f = _gate_weights(p['enc_frag'])
    t_f = _k_mm(hnode_f, wcat_f[0:H], bcat_f)
    h1_f = _msg_pass(_sc_gather(t_f, src_t), pos_t,
                     _pad_rows(wcat_f[H:H + MAX_POS], 24),
                     bg_t, wcat_f, H + MAX_POS, mtp)
    wo_f, bo_f = p['enc_frag']['Wo']
    p_f = _k_mm(hnode_f, wo_f[0:H], jnp.zeros((H,), F32))
    nn_f = _sc_gather_nbr([h1_f], ag_t)[0]
    hinter_p = _k_node(p_f, nn_f, wo_f[H:], bo_f, 4, True)

    w_c, b_c = p['W_c']
    ec_tab = _k_mm(_pad_rows(p['E_c'], _ceil_to(p['E_c'].shape[0], BM)),
                   w_c[0:H], jnp.zeros((H,), F32))
    eg_c = _sc_gather(ec_tab, tf0)
    hnode_c = _k_hc(eg_c, hinter_p, hbond_p, w_c[H:2 * H], w_c[2 * H:], b_c)
    wcat_c, bcat_c = _gate_weights(p['enc_inter'])
    t_c = _k_mm(hnode_c, wcat_c[0:H], bcat_c)
    h1_c = _msg_pass(_sc_gather(t_c, src_t), pos_t,
                     _pad_rows(wcat_c[H:H + MAX_POS], 24),
                     bg_t, wcat_c, H + MAX_POS, mtp)
    wo_c, bo_c = p['enc_inter']['Wo']
    p_c = _k_mm(hnode_c, wo_c[0:H], jnp.zeros((H,), F32))
    nn_c = _sc_gather_nbr([h1_c], ag_t)[0]
    hnode_p = _k_node(p_c, nn_c, wo_c[H:], bo_c, 4, True)

    wr, br = p['W_root']
    roots_p = _pad_idx(roots, 256)
    fr = _sc_gather(hnode_c, roots_p)[:nroot]
    agr = _pad_idx(jnp.take(tree_agraph, roots, axis=0).reshape(-1), 256)
    rn = _sc_gather(h1_c, agr)[:nroot * 4].reshape(nroot, 4 * H)
    hroot = _k_root(fr, rn, wr[0:H], wr[H:], br)

    return (hroot, hnode_p[:nt], hinter_p[:nt], hbond_p[:nt], hatom_p[:ng])


# speedup vs baseline: 2.1008x; 1.0615x over previous
import functools

import jax
import jax.numpy as jnp
from jax import lax
from jax.experimental import pallas as pl
from jax.experimental.pallas import tpu as pltpu, tpu_sc as plsc

H = 128
MAX_POS = 20
ATOM_SIZE = 40
NBOND = 4
NW = 32
BM = 512
F32 = jnp.float32


def _pick_chunk(bpw, d):
    cap = max(8, min(bpw, (128 * 1024) // (d * 4)))
    for c in range(cap - cap % 8, 7, -8):
        if bpw % c == 0:
            return c
    return 8


def _pick_chunk2(bpw, d, a_dim, ntab):
    cap = max(8, min(bpw, 470_000 // (2 * ntab * a_dim * d * 4)))
    best = 8
    for c in range(8, cap + 1, 8):
        if bpw % c == 0:
            best = c
    return best


def _sc_gather_pipe(tables, idx_t, d):
    a_dim, mp = idx_t.shape
    ntab = len(tables)
    bpw = mp // NW
    chunk = _pick_chunk2(bpw, d, a_dim, ntab)
    nloop = bpw // chunk
    k_idx = a_dim * chunk
    idx_p = idx_t.reshape(a_dim, NW, nloop, chunk).transpose(1, 2, 0, 3).reshape(-1)
    mesh = plsc.VectorSubcoreMesh(core_axis_name="c", subcore_axis_name="s")
    shp = jax.ShapeDtypeStruct((a_dim * mp, d), F32)

    @functools.partial(
        pl.kernel,
        out_type=tuple(shp for _ in range(ntab)),
        mesh=mesh,
        scratch_types=[
            pltpu.VMEM((k_idx,), jnp.int32),
            pltpu.VMEM((k_idx,), jnp.int32),
            pltpu.VMEM((ntab * k_idx, d), F32),
            pltpu.VMEM((ntab * k_idx, d), F32),
            pltpu.SemaphoreType.DMA,
            pltpu.SemaphoreType.DMA,
            pltpu.SemaphoreType.DMA,
            pltpu.SemaphoreType.DMA,
            pltpu.SemaphoreType.DMA,
        ],
    )
    def k(*refs):
        tabs = refs[:ntab]
        idx_hbm = refs[ntab]
        outs = refs[ntab + 1:ntab + 1 + ntab]
        (i0, i1, r0, r1, is0, is1, gsem, ss0, ss1) = refs[ntab + 1 + ntab:]
        idxb, rowsb, isems, ssems = [i0, i1], [r0, r1], [is0, is1], [ss0, ss1]
        wid = lax.axis_index("s") * 2 + lax.axis_index("c")
        ibase = wid * nloop * k_idx
        obase = wid * bpw

        def idx_src(j):
            return idx_hbm.at[pl.ds(ibase + j * k_idx, k_idx)]

        def out_reg(t, a, j):
            return outs[t].at[pl.ds(a * mp + obase + j * chunk, chunk)]

        def rslice(rb, t, a):
            return rb.at[pl.ds((t * a_dim + a) * chunk, chunk)]

        def _when(cond, fn):
            if isinstance(cond, bool):
                if cond:
                    fn()
            else:
                pl.when(cond)(fn)

        def step(j, b):
            def drain_prev():
                for t in range(ntab):
                    for a in range(a_dim):
                        pltpu.make_async_copy(
                            rslice(rowsb[b], t, a), out_reg(t, a, j - 2),
                            ssems[b]).wait()
            _when(j >= 2, drain_prev)
            pltpu.make_async_copy(idx_src(j), idxb[b], isems[b]).wait()
            descs = [
                pltpu.async_copy(
                    tabs[t].at[idxb[b]],
                    rowsb[b].at[pl.ds(t * k_idx, k_idx)], gsem)
                for t in range(ntab)
            ]
            for de in descs:
                de.wait()

            def prefetch():
                pltpu.async_copy(idx_src(j + 2), idxb[b], isems[b])
            _when(j + 2 < nloop, prefetch)
            for t in range(ntab):
                for a in range(a_dim):
                    pltpu.async_copy(rslice(rowsb[b], t, a), out_reg(t, a, j),
                                     ssems[b])

        pltpu.async_copy(idx_src(0), idxb[0], isems[0])
        if nloop > 1:
            pltpu.async_copy(idx_src(1), idxb[1], isems[1])

        def body2(j0, carry):
            step(j0 * 2, 0)
            step(j0 * 2 + 1, 1)
            return carry

        lax.fori_loop(0, nloop // 2, body2, 0)
        if nloop % 2:
            step(nloop - 1, (nloop - 1) % 2)
        for jl in range(max(0, nloop - 2), nloop):
            b = jl % 2
            for t in range(ntab):
                for a in range(a_dim):
                    pltpu.make_async_copy(
                        rslice(rowsb[b], t, a), out_reg(t, a, jl),
                        ssems[b]).wait()

    out = k(*tables, idx_p)
    return tuple(out) if isinstance(out, (list, tuple)) else (out,)


def _sc_gather(table, idx):
    return _sc_gather_pipe([table], idx[None, :], table.shape[1])[0]


def _sc_gather_nbr(tables, idx_t):
    a_dim, mp = idx_t.shape
    outs = _sc_gather_pipe(tables, idx_t, H)
    return tuple(o.reshape(a_dim, mp, H) for o in outs)


def _row(d):
    return pl.BlockSpec((BM, d), lambda i: (i, 0))


def _full(shape):
    return pl.BlockSpec(shape, lambda i: tuple(0 for _ in shape))


def _rowmask(x):
    rows = pl.program_id(0) * BM + lax.broadcasted_iota(jnp.int32, (BM, 1), 0)
    return x * (rows != 0).astype(F32)


def _dot(a, b):
    return jnp.dot(a, b, preferred_element_type=F32)


def _sigm(x):
    return jax.nn.sigmoid(x)


def _k_mm(x, w, b, relu=False):
    m, kdim = x.shape
    n = w.shape[1]

    def body(x_ref, w_ref, b_ref, o_ref):
        acc = _dot(x_ref[...], w_ref[...]) + b_ref[...]
        o_ref[...] = jnp.maximum(acc, 0.0) if relu else acc

    return pl.pallas_call(
        body,
        grid=(m // BM,),
        in_specs=[_row(kdim), _full((kdim, n)), _full((1, n))],
        out_specs=_row(n),
        out_shape=jax.ShapeDtypeStruct((m, n), F32),
    )(x, w, b.reshape(1, n))


def _xparts(xg, pos_ref, ptab_ref, bond_ref, btab_ref, pw, bw):
    oh_p = (pos_ref[...] == lax.broadcasted_iota(jnp.int32, (BM, pw), 1)
            ).astype(F32)
    x = xg + _dot(oh_p, ptab_ref[...])
    if btab_ref is not None:
        oh_b = (bond_ref[...] == lax.broadcasted_iota(jnp.int32, (BM, bw), 1)
                ).astype(F32)
        x = x + _dot(oh_b, btab_ref[...])
    return x


def _k_d0(xg, pos, ptab, wfh, bond=None, btab=None):
    m = xg.shape[0]
    pw = ptab.shape[0]
    bw = btab.shape[0] if btab is not None else 0

    def body(*refs):
        if btab is None:
            xg_ref, pos_ref, ptab_ref, wfh_ref, hgc_ref = refs
            bond_ref = btab_ref = None
        else:
            (xg_ref, pos_ref, bond_ref, ptab_ref, btab_ref, wfh_ref,
             hgc_ref) = refs
        xv = _xparts(xg_ref[...], pos_ref, ptab_ref, bond_ref, btab_ref, pw, bw)
        i = _sigm(xv[:, 0:H])
        o = _sigm(xv[:, H:2 * H])
        u = jnp.tanh(xv[:, 2 * H:3 * H])
        c = _rowmask(i * u)
        h = _rowmask(o * jnp.tanh(c))
        hgc_ref[...] = jnp.concatenate([h, _dot(h, wfh_ref[...]), c], axis=-1)

    specs = [_row(4 * H), pl.BlockSpec((BM, 1), lambda i: (i, 0))]
    args = [xg, pos.reshape(m, 1)]
    if btab is not None:
        specs.append(pl.BlockSpec((BM, 1), lambda i: (i, 0)))
        args.append(bond.reshape(m, 1))
    specs.append(_full(ptab.shape))
    args.append(ptab)
    if btab is not None:
        specs.append(_full(btab.shape))
        args.append(btab)
    specs.append(_full((H, H)))
    args.append(wfh)

    return pl.pallas_call(
        body,
        grid=(m // BM,),
        in_specs=specs,
        out_specs=_row(3 * H),
        out_shape=jax.ShapeDtypeStruct((m, 3 * H), F32),
    )(*args)


def _k_d1(xg, pos, ptab, hgcn, wh3, bond=None, btab=None):
    m = xg.shape[0]
    pw = ptab.shape[0]
    bw = btab.shape[0] if btab is not None else 0
    nbr = pl.BlockSpec((4, BM, 3 * H), lambda i: (0, i, 0))

    def body(*refs):
        if btab is None:
            (xg_ref, pos_ref, ptab_ref, hgcn_ref, wh3_ref, h_ref) = refs
            bond_ref = btab_ref = None
        else:
            (xg_ref, pos_ref, bond_ref, ptab_ref, btab_ref, hgcn_ref,
             wh3_ref, h_ref) = refs
        xv = _xparts(xg_ref[...], pos_ref, ptab_ref, bond_ref, btab_ref, pw, bw)
        hs = (hgcn_ref[0, :, 0:H] + hgcn_ref[1, :, 0:H]
              + hgcn_ref[2, :, 0:H] + hgcn_ref[3, :, 0:H])
        z = _dot(hs, wh3_ref[...])
        i = _sigm(xv[:, 0:H] + z[:, 0:H])
        o = _sigm(xv[:, H:2 * H] + z[:, H:2 * H])
        u = jnp.tanh(xv[:, 2 * H:3 * H] + z[:, 2 * H:3 * H])
        xf = xv[:, 3 * H:4 * H]
        fc = jnp.zeros((BM, H), F32)
        for a in range(4):
            fc = fc + (_sigm(xf + hgcn_ref[a, :, H:2 * H])
                       * hgcn_ref[a, :, 2 * H:3 * H])
        c = i * u + fc
        h_ref[...] = _rowmask(o * jnp.tanh(c))

    specs = [_row(4 * H), pl.BlockSpec((BM, 1), lambda i: (i, 0))]
    args = [xg, pos.reshape(m, 1)]
    if btab is not None:
        specs.append(pl.BlockSpec((BM, 1), lambda i: (i, 0)))
        args.append(bond.reshape(m, 1))
    specs.append(_full(ptab.shape))
    args.append(ptab)
    if btab is not None:
        specs.append(_full(btab.shape))
        args.append(btab)
    specs += [nbr, _full((H, 3 * H))]
    args += [hgcn, wh3]

    return pl.pallas_call(
        body,
        grid=(m // BM,),
        in_specs=specs,
        out_specs=_row(H),
        out_shape=jax.ShapeDtypeStruct((m, H), F32),
    )(*args)


def _k_node(p, nn, w, b, na, mask0):
    m = p.shape[0]

    def body(p_ref, nn_ref, w_ref, b_ref, o_ref):
        hs = jnp.zeros((BM, H), F32)
        for a in range(na):
            hs = hs + nn_ref[a]
        out = jnp.maximum(p_ref[...] + _dot(hs, w_ref[...]) + b_ref[...], 0.0)
        o_ref[...] = _rowmask(out) if mask0 else out

    return pl.pallas_call(
        body,
        grid=(m // BM,),
        in_specs=[_row(H), pl.BlockSpec((na, BM, H), lambda i: (0, i, 0)),
                  _full((H, H)), _full((1, H))],
        out_specs=_row(H),
        out_shape=jax.ShapeDtypeStruct((m, H), F32),
    )(p, nn, w, b.reshape(1, H))


def _k_hc(eg, hi, hb, w1, w2, b):
    m = eg.shape[0]

    def body(eg_ref, hi_ref, hb_ref, w1_ref, w2_ref, b_ref, o_ref):
        o_ref[...] = jnp.maximum(
            eg_ref[...] + _dot(hi_ref[...], w1_ref[...])
            + _dot(hb_ref[...], w2_ref[...]) + b_ref[...], 0.0)

    return pl.pallas_call(
        body,
        grid=(m // BM,),
        in_specs=[_row(H), _row(H), _row(H), _full((H, H)), _full((H, H)),
                  _full((1, H))],
        out_specs=_row(H),
        out_shape=jax.ShapeDtypeStruct((m, H), F32),
    )(eg, hi, hb, w1, w2, b.reshape(1, H))


def _k_root(fr, rn, w1, w2, b):
    nr = fr.shape[0]

    def body(fr_ref, rn_ref, w1_ref, w2_ref, b_ref, o_ref):
        rnv = rn_ref[...]
        hs = (rnv[:, 0:H] + rnv[:, H:2 * H]
              + rnv[:, 2 * H:3 * H] + rnv[:, 3 * H:4 * H])
        o_ref[...] = jnp.tanh(_dot(fr_ref[...], w1_ref[...])
                              + _dot(hs, w2_ref[...]) + b_ref[...])

    return pl.pallas_call(
        body,
        grid=(1,),
        in_specs=[_full((nr, H)), _full((nr, 4 * H)), _full((H, H)),
                  _full((H, H)), _full((1, H))],
        out_specs=_full((nr, H)),
        out_shape=jax.ShapeDtypeStruct((nr, H), F32),
    )(fr, rn, w1, w2, b.reshape(1, H))


def _ceil_to(x, m):
    return (x + m - 1) // m * m


def _pad_rows(x, n):
    return jnp.pad(x, ((0, n - x.shape[0]),) + ((0, 0),) * (x.ndim - 1))


def _pad_idx(idx, n):
    return jnp.pad(idx.astype(jnp.int32), (0, n - idx.shape[0]))


def _gate_weights(p):
    wcat = jnp.concatenate(
        [p['Wi'][0], p['Wog'][0], p['Wu'][0], p['Wf'][0]], axis=1)
    bcat = jnp.concatenate(
        [p['Wi'][1], p['Wog'][1], p['Wu'][1], p['Wf'][1]])
    return wcat, bcat


def _pad_idx2(idx, n):
    return jnp.pad(idx.astype(jnp.int32), ((0, 0), (0, n - idx.shape[1])))


def _msg_pass(xg, pos, ptab, bg_t, wcat, din, mp, bond=None, btab=None):
    wfh = wcat[din:, 3 * H:4 * H]
    wh3 = wcat[din:, 0:3 * H]
    hgc0 = _k_d0(xg, pos, ptab, wfh, bond, btab)
    hgcn = _sc_gather_pipe([hgc0], bg_t, 3 * H)[0].reshape(4, mp, 3 * H)
    return _k_d1(xg, pos, ptab, hgcn, wh3, bond, btab)


def kernel(params, tree_fnode, tree_fmess, tree_agraph, tree_bgraph,
           tree_cgraph, roots, graph_fnode, graph_fmess, graph_agraph,
           graph_bgraph):
    p = params
    nt, mt = tree_fnode.shape[0], tree_fmess.shape[0]
    ng, mg = graph_fnode.shape[0], graph_fmess.shape[0]
    cl = tree_cgraph.shape[1]
    nroot = roots.shape[0]
    ntp, mtp = _ceil_to(nt, 2048), _ceil_to(mt, 2048)
    ngp, mgp = _ceil_to(ng, 2048), _ceil_to(mg, 2048)

    fnode_g_p = _pad_idx(graph_fnode, ngp)
    src_g = _pad_idx(graph_fmess[:, 0], mgp)
    bond_g = _pad_idx(graph_fmess[:, 2], mgp)
    pos_g = _pad_idx(graph_fmess[:, 3], mgp)
    bg_g = _pad_idx2(graph_bgraph.T, mgp)
    ag_g = _pad_idx2(graph_agraph.T, ngp)
    src_t = _pad_idx(tree_fmess[:, 0], mtp)
    pos_t = _pad_idx(tree_fmess[:, 2], mtp)
    bg_t = _pad_idx2(tree_bgraph.T, mtp)
    ag_t = _pad_idx2(tree_agraph.T, ntp)
    cg_t = _pad_idx2(tree_cgraph.T, ntp)
    tf0 = _pad_idx(tree_fnode[:, 0], ntp)
    tf1 = _pad_idx(tree_fnode[:, 1], ntp)
    tf2 = _pad_idx(tree_fnode[:, 2], ntp)

    wcat_g, bcat_g = _gate_weights(p['enc_graph'])
    atom_tab = wcat_g[0:ATOM_SIZE]
    bond_tab = _pad_rows(wcat_g[ATOM_SIZE:ATOM_SIZE + NBOND], 8)
    pos_tab_g = _pad_rows(wcat_g[ATOM_SIZE + NBOND:ATOM_SIZE + NBOND + MAX_POS]
                          + bcat_g[None, :], 24)
    atom_rows = _sc_gather(atom_tab, fnode_g_p)
    xg_g = _sc_gather(atom_rows, src_g)
    h1_g = _msg_pass(xg_g, pos_g, pos_tab_g, bg_g, wcat_g,
                     ATOM_SIZE + NBOND + MAX_POS, mgp, bond_g, bond_tab)
    wo_g, bo_g = p['enc_graph']['Wo']
    p_g = _sc_gather(wo_g[0:ATOM_SIZE], fnode_g_p)
    nn_g = _sc_gather_nbr([h1_g], ag_g)[0]
    hatom_p = _k_node(p_g, nn_g, wo_g[ATOM_SIZE:], bo_g, 4, True)

    wcat_b, bcat_b = _gate_weights(p['enc_bond'])
    hnode_b = _sc_gather(p['E_l'], tf2)
    t_b = _k_mm(hnode_b, wcat_b[0:H], bcat_b)
    h1_b = _msg_pass(_sc_gather(t_b, src_t), pos_t,
                     _pad_rows(wcat_b[H:H + MAX_POS], 24),
                     bg_t, wcat_b, H + MAX_POS, mtp)
    wo_b, bo_b = p['enc_bond']['Wo']
    p_b = _k_mm(hnode_b, wo_b[0:H], jnp.zeros((H,), F32))
    nn_b = _sc_gather_nbr([h1_b], ag_t)[0]
    hbond_p = _k_node(p_b, nn_b, wo_b[H:], bo_b, 4, True)

    w_i, b_i = p['W_i']
    ei_tab = _k_mm(_pad_rows(p['E_i'], _ceil_to(p['E_i'].shape[0], BM)),
                   w_i[0:H], jnp.zeros((H,), F32))
    eg_f = _sc_gather(ei_tab, tf1)
    cl_rows = _sc_gather_nbr([hatom_p], cg_t)[0]
    hnode_f = _k_node(eg_f, cl_rows, w_i[H:], b_i, cl, False)
    wcat_f, bcat_f = _gate_weights(p['enc_frag'])
    t_f = _k_mm(hnode_f, wcat_f[0:H], bcat_f)
    h1_f = _msg_pass(_sc_gather(t_f, src_t), pos_t,
                     _pad_rows(wcat_f[H:H + MAX_POS], 24),
                     bg_t, wcat_f, H + MAX_POS, mtp)
    wo_f, bo_f = p['enc_frag']['Wo']
    p_f = _k_mm(hnode_f, wo_f[0:H], jnp.zeros((H,), F32))
    nn_f = _sc_gather_nbr([h1_f], ag_t)[0]
    hinter_p = _k_node(p_f, nn_f, wo_f[H:], bo_f, 4, True)

    w_c, b_c = p['W_c']
    ec_tab = _k_mm(_pad_rows(p['E_c'], _ceil_to(p['E_c'].shape[0], BM)),
                   w_c[0:H], jnp.zeros((H,), F32))
    eg_c = _sc_gather(ec_tab, tf0)
    hnode_c = _k_hc(eg_c, hinter_p, hbond_p, w_c[H:2 * H], w_c[2 * H:], b_c)
    wcat_c, bcat_c = _gate_weights(p['enc_inter'])
    t_c = _k_mm(hnode_c, wcat_c[0:H], bcat_c)
    h1_c = _msg_pass(_sc_gather(t_c, src_t), pos_t,
                     _pad_rows(wcat_c[H:H + MAX_POS], 24),
                     bg_t, wcat_c, H + MAX_POS, mtp)
    wo_c, bo_c = p['enc_inter']['Wo']
    p_c = _k_mm(hnode_c, wo_c[0:H], jnp.zeros((H,), F32))
    nn_c = _sc_gather_nbr([h1_c], ag_t)[0]
    hnode_p = _k_node(p_c, nn_c, wo_c[H:], bo_c, 4, True)

    wr, br = p['W_root']
    roots_p = _pad_idx(roots, 256)
    fr = _sc_gather(hnode_c, roots_p)[:nroot]
    agr = _pad_idx(jnp.take(tree_agraph, roots, axis=0).reshape(-1), 256)
    rn = _sc_gather(h1_c, agr)[:nroot * 4].reshape(nroot, 4 * H)
    hroot = _k_root(fr, rn, wr[0:H], wr[H:], br)

    return (hroot, hnode_p[:nt], hinter_p[:nt], hbond_p[:nt], hatom_p[:ng])


# speedup vs baseline: 2.1740x; 1.0348x over previous
import functools

import jax
import jax.numpy as jnp
from jax import lax
from jax.experimental import pallas as pl
from jax.experimental.pallas import tpu as pltpu, tpu_sc as plsc

H = 128
MAX_POS = 20
ATOM_SIZE = 40
NBOND = 4
NW = 32
BM = 512
F32 = jnp.float32


def _pick_chunk(bpw, d):
    cap = max(8, min(bpw, (128 * 1024) // (d * 4)))
    for c in range(cap - cap % 8, 7, -8):
        if bpw % c == 0:
            return c
    return 8


def _pick_chunk2(bpw, d, a_dim, ntab):
    cap = max(8, min(bpw, 470_000 // (2 * ntab * a_dim * d * 4)))
    best = 8
    for c in range(8, cap + 1, 8):
        if bpw % c == 0:
            best = c
    return best


def _sc_gather_pipe(tables, idx_t, d):
    a_dim, mp = idx_t.shape
    ntab = len(tables)
    bpw = mp // NW
    chunk = _pick_chunk2(bpw, d, a_dim, ntab)
    nloop = bpw // chunk
    k_idx = a_dim * chunk
    idx_p = idx_t.reshape(a_dim, NW, nloop, chunk).transpose(1, 2, 0, 3).reshape(-1)
    mesh = plsc.VectorSubcoreMesh(core_axis_name="c", subcore_axis_name="s")
    shp = jax.ShapeDtypeStruct((a_dim * mp, d), F32)

    @functools.partial(
        pl.kernel,
        out_type=tuple(shp for _ in range(ntab)),
        mesh=mesh,
        scratch_types=[
            pltpu.VMEM((k_idx,), jnp.int32),
            pltpu.VMEM((k_idx,), jnp.int32),
            pltpu.VMEM((ntab * k_idx, d), F32),
            pltpu.VMEM((ntab * k_idx, d), F32),
            pltpu.SemaphoreType.DMA,
            pltpu.SemaphoreType.DMA,
            pltpu.SemaphoreType.DMA,
            pltpu.SemaphoreType.DMA,
            pltpu.SemaphoreType.DMA,
        ],
    )
    def k(*refs):
        tabs = refs[:ntab]
        idx_hbm = refs[ntab]
        outs = refs[ntab + 1:ntab + 1 + ntab]
        (i0, i1, r0, r1, is0, is1, gsem, ss0, ss1) = refs[ntab + 1 + ntab:]
        idxb, rowsb, isems, ssems = [i0, i1], [r0, r1], [is0, is1], [ss0, ss1]
        wid = lax.axis_index("s") * 2 + lax.axis_index("c")
        ibase = wid * nloop * k_idx
        obase = wid * bpw

        def idx_src(j):
            return idx_hbm.at[pl.ds(ibase + j * k_idx, k_idx)]

        def out_reg(t, a, j):
            return outs[t].at[pl.ds(a * mp + obase + j * chunk, chunk)]

        def rslice(rb, t, a):
            return rb.at[pl.ds((t * a_dim + a) * chunk, chunk)]

        def _when(cond, fn):
            if isinstance(cond, bool):
                if cond:
                    fn()
            else:
                pl.when(cond)(fn)

        def step(j, b):
            def drain_prev():
                for t in range(ntab):
                    for a in range(a_dim):
                        pltpu.make_async_copy(
                            rslice(rowsb[b], t, a), out_reg(t, a, j - 2),
                            ssems[b]).wait()
            _when(j >= 2, drain_prev)
            pltpu.make_async_copy(idx_src(j), idxb[b], isems[b]).wait()
            descs = [
                pltpu.async_copy(
                    tabs[t].at[idxb[b]],
                    rowsb[b].at[pl.ds(t * k_idx, k_idx)], gsem)
                for t in range(ntab)
            ]
            for de in descs:
                de.wait()

            def prefetch():
                pltpu.async_copy(idx_src(j + 2), idxb[b], isems[b])
            _when(j + 2 < nloop, prefetch)
            for t in range(ntab):
                for a in range(a_dim):
                    pltpu.async_copy(rslice(rowsb[b], t, a), out_reg(t, a, j),
                                     ssems[b])

        pltpu.async_copy(idx_src(0), idxb[0], isems[0])
        if nloop > 1:
            pltpu.async_copy(idx_src(1), idxb[1], isems[1])

        def body2(j0, carry):
            step(j0 * 2, 0)
            step(j0 * 2 + 1, 1)
            return carry

        lax.fori_loop(0, nloop // 2, body2, 0)
        if nloop % 2:
            step(nloop - 1, (nloop - 1) % 2)
        for jl in range(max(0, nloop - 2), nloop):
            b = jl % 2
            for t in range(ntab):
                for a in range(a_dim):
                    pltpu.make_async_copy(
                        rslice(rowsb[b], t, a), out_reg(t, a, jl),
                        ssems[b]).wait()

    out = k(*tables, idx_p)
    return tuple(out) if isinstance(out, (list, tuple)) else (out,)


def _sc_gather_sum(table, idx_t):
    a_dim, mp = idx_t.shape
    bpw = mp // NW
    chunk = _pick_chunk2(bpw, H, a_dim, 1)
    nloop = bpw // chunk
    k_idx = a_dim * chunk
    idx_p = idx_t.reshape(a_dim, NW, nloop, chunk).transpose(1, 2, 0, 3).reshape(-1)
    mesh = plsc.VectorSubcoreMesh(core_axis_name="c", subcore_axis_name="s")

    @functools.partial(
        pl.kernel,
        out_type=jax.ShapeDtypeStruct((mp, H), F32),
        mesh=mesh,
        scratch_types=[
            pltpu.VMEM((k_idx,), jnp.int32),
            pltpu.VMEM((k_idx,), jnp.int32),
            pltpu.VMEM((k_idx, H), F32),
            pltpu.VMEM((k_idx, H), F32),
            pltpu.VMEM((chunk, H), F32),
            pltpu.VMEM((chunk, H), F32),
            pltpu.SemaphoreType.DMA,
            pltpu.SemaphoreType.DMA,
            pltpu.SemaphoreType.DMA,
            pltpu.SemaphoreType.DMA,
            pltpu.SemaphoreType.DMA,
        ],
    )
    def k(tab_hbm, idx_hbm, out_hbm, i0, i1, r0, r1, h0, h1,
          is0, is1, gsem, ss0, ss1):
        idxb, rowsb, hsb = [i0, i1], [r0, r1], [h0, h1]
        isems, ssems = [is0, is1], [ss0, ss1]
        wid = lax.axis_index("s") * 2 + lax.axis_index("c")
        ibase = wid * nloop * k_idx
        obase = wid * bpw

        def idx_src(j):
            return idx_hbm.at[pl.ds(ibase + j * k_idx, k_idx)]

        def out_reg(j):
            return out_hbm.at[pl.ds(obase + j * chunk, chunk)]

        def _when(cond, fn):
            if isinstance(cond, bool):
                if cond:
                    fn()
            else:
                pl.when(cond)(fn)

        def step(j, b):
            def drain_prev():
                pltpu.make_async_copy(hsb[b], out_reg(j - 2), ssems[b]).wait()
            _when(j >= 2, drain_prev)
            pltpu.make_async_copy(idx_src(j), idxb[b], isems[b]).wait()
            pltpu.async_copy(tab_hbm.at[idxb[b]], rowsb[b], gsem).wait()

            def prefetch():
                pltpu.async_copy(idx_src(j + 2), idxb[b], isems[b])
            _when(j + 2 < nloop, prefetch)

            def red(m, carry):
                for v in range(H // 16):
                    acc = rowsb[b][m, pl.ds(v * 16, 16)]
                    for a in range(1, a_dim):
                        acc = acc + rowsb[b][a * chunk + m, pl.ds(v * 16, 16)]
                    hsb[b][m, pl.ds(v * 16, 16)] = acc
                return carry

            lax.fori_loop(0, chunk, red, 0)
            pltpu.async_copy(hsb[b], out_reg(j), ssems[b])

        pltpu.async_copy(idx_src(0), idxb[0], isems[0])
        if nloop > 1:
            pltpu.async_copy(idx_src(1), idxb[1], isems[1])

        def body2(j0, carry):
            step(j0 * 2, 0)
            step(j0 * 2 + 1, 1)
            return carry

        lax.fori_loop(0, nloop // 2, body2, 0)
        if nloop % 2:
            step(nloop - 1, (nloop - 1) % 2)
        for jl in range(max(0, nloop - 2), nloop):
            pltpu.make_async_copy(hsb[jl % 2], out_reg(jl), ssems[jl % 2]).wait()

    return k(table, idx_p)


def _sc_gather(table, idx):
    return _sc_gather_pipe([table], idx[None, :], table.shape[1])[0]


def _sc_gather_nbr(tables, idx_t):
    a_dim, mp = idx_t.shape
    outs = _sc_gather_pipe(tables, idx_t, H)
    return tuple(o.reshape(a_dim, mp, H) for o in outs)


def _row(d):
    return pl.BlockSpec((BM, d), lambda i: (i, 0))


def _full(shape):
    return pl.BlockSpec(shape, lambda i: tuple(0 for _ in shape))


def _rowmask(x):
    rows = pl.program_id(0) * BM + lax.broadcasted_iota(jnp.int32, (BM, 1), 0)
    return x * (rows != 0).astype(F32)


def _dot(a, b):
    return jnp.dot(a, b, preferred_element_type=F32)


def _sigm(x):
    return jax.nn.sigmoid(x)


def _k_mm(x, w, b, relu=False):
    m, kdim = x.shape
    n = w.shape[1]

    def body(x_ref, w_ref, b_ref, o_ref):
        acc = _dot(x_ref[...], w_ref[...]) + b_ref[...]
        o_ref[...] = jnp.maximum(acc, 0.0) if relu else acc

    return pl.pallas_call(
        body,
        grid=(m // BM,),
        in_specs=[_row(kdim), _full((kdim, n)), _full((1, n))],
        out_specs=_row(n),
        out_shape=jax.ShapeDtypeStruct((m, n), F32),
    )(x, w, b.reshape(1, n))


def _xparts(xg, pos_ref, ptab_ref, bond_ref, btab_ref, pw, bw):
    oh_p = (pos_ref[...] == lax.broadcasted_iota(jnp.int32, (BM, pw), 1)
            ).astype(F32)
    x = xg + _dot(oh_p, ptab_ref[...])
    if btab_ref is not None:
        oh_b = (bond_ref[...] == lax.broadcasted_iota(jnp.int32, (BM, bw), 1)
                ).astype(F32)
        x = x + _dot(oh_b, btab_ref[...])
    return x


def _k_d0(xg, pos, ptab, wfh, bond=None, btab=None):
    m = xg.shape[0]
    pw = ptab.shape[0]
    bw = btab.shape[0] if btab is not None else 0

    def body(*refs):
        if btab is None:
            xg_ref, pos_ref, ptab_ref, wfh_ref, hgc_ref = refs
            bond_ref = btab_ref = None
        else:
            (xg_ref, pos_ref, bond_ref, ptab_ref, btab_ref, wfh_ref,
             hgc_ref) = refs
        xv = _xparts(xg_ref[...], pos_ref, ptab_ref, bond_ref, btab_ref, pw, bw)
        i = _sigm(xv[:, 0:H])
        o = _sigm(xv[:, H:2 * H])
        u = jnp.tanh(xv[:, 2 * H:3 * H])
        c = _rowmask(i * u)
        h = _rowmask(o * jnp.tanh(c))
        hgc_ref[...] = jnp.concatenate([h, _dot(h, wfh_ref[...]), c], axis=-1)

    specs = [_row(4 * H), pl.BlockSpec((BM, 1), lambda i: (i, 0))]
    args = [xg, pos.reshape(m, 1)]
    if btab is not None:
        specs.append(pl.BlockSpec((BM, 1), lambda i: (i, 0)))
        args.append(bond.reshape(m, 1))
    specs.append(_full(ptab.shape))
    args.append(ptab)
    if btab is not None:
        specs.append(_full(btab.shape))
        args.append(btab)
    specs.append(_full((H, H)))
    args.append(wfh)

    return pl.pallas_call(
        body,
        grid=(m // BM,),
        in_specs=specs,
        out_specs=_row(3 * H),
        out_shape=jax.ShapeDtypeStruct((m, 3 * H), F32),
    )(*args)


def _k_d1(xg, pos, ptab, hgcn, wh3, bond=None, btab=None):
    m = xg.shape[0]
    pw = ptab.shape[0]
    bw = btab.shape[0] if btab is not None else 0
    nbr = pl.BlockSpec((4, BM, 3 * H), lambda i: (0, i, 0))

    def body(*refs):
        if btab is None:
            (xg_ref, pos_ref, ptab_ref, hgcn_ref, wh3_ref, h_ref) = refs
            bond_ref = btab_ref = None
        else:
            (xg_ref, pos_ref, bond_ref, ptab_ref, btab_ref, hgcn_ref,
             wh3_ref, h_ref) = refs
        xv = _xparts(xg_ref[...], pos_ref, ptab_ref, bond_ref, btab_ref, pw, bw)
        hs = (hgcn_ref[0, :, 0:H] + hgcn_ref[1, :, 0:H]
              + hgcn_ref[2, :, 0:H] + hgcn_ref[3, :, 0:H])
        z = _dot(hs, wh3_ref[...])
        i = _sigm(xv[:, 0:H] + z[:, 0:H])
        o = _sigm(xv[:, H:2 * H] + z[:, H:2 * H])
        u = jnp.tanh(xv[:, 2 * H:3 * H] + z[:, 2 * H:3 * H])
        xf = xv[:, 3 * H:4 * H]
        fc = jnp.zeros((BM, H), F32)
        for a in range(4):
            fc = fc + (_sigm(xf + hgcn_ref[a, :, H:2 * H])
                       * hgcn_ref[a, :, 2 * H:3 * H])
        c = i * u + fc
        h_ref[...] = _rowmask(o * jnp.tanh(c))

    specs = [_row(4 * H), pl.BlockSpec((BM, 1), lambda i: (i, 0))]
    args = [xg, pos.reshape(m, 1)]
    if btab is not None:
        specs.append(pl.BlockSpec((BM, 1), lambda i: (i, 0)))
        args.append(bond.reshape(m, 1))
    specs.append(_full(ptab.shape))
    args.append(ptab)
    if btab is not None:
        specs.append(_full(btab.shape))
        args.append(btab)
    specs += [nbr, _full((H, 3 * H))]
    args += [hgcn, wh3]

    return pl.pallas_call(
        body,
        grid=(m // BM,),
        in_specs=specs,
        out_specs=_row(H),
        out_shape=jax.ShapeDtypeStruct((m, H), F32),
    )(*args)


def _k_node(p, hs, w, b, mask0):
    m = p.shape[0]

    def body(p_ref, hs_ref, w_ref, b_ref, o_ref):
        out = jnp.maximum(
            p_ref[...] + _dot(hs_ref[...], w_ref[...]) + b_ref[...], 0.0)
        o_ref[...] = _rowmask(out) if mask0 else out

    return pl.pallas_call(
        body,
        grid=(m // BM,),
        in_specs=[_row(H), _row(H), _full((H, H)), _full((1, H))],
        out_specs=_row(H),
        out_shape=jax.ShapeDtypeStruct((m, H), F32),
    )(p, hs, w, b.reshape(1, H))


def _k_hc(eg, hi, hb, w1, w2, b):
    m = eg.shape[0]

    def body(eg_ref, hi_ref, hb_ref, w1_ref, w2_ref, b_ref, o_ref):
        o_ref[...] = jnp.maximum(
            eg_ref[...] + _dot(hi_ref[...], w1_ref[...])
            + _dot(hb_ref[...], w2_ref[...]) + b_ref[...], 0.0)

    return pl.pallas_call(
        body,
        grid=(m // BM,),
        in_specs=[_row(H), _row(H), _row(H), _full((H, H)), _full((H, H)),
                  _full((1, H))],
        out_specs=_row(H),
        out_shape=jax.ShapeDtypeStruct((m, H), F32),
    )(eg, hi, hb, w1, w2, b.reshape(1, H))


def _k_root(fr, rn, w1, w2, b):
    nr = fr.shape[0]

    def body(fr_ref, rn_ref, w1_ref, w2_ref, b_ref, o_ref):
        rnv = rn_ref[...]
        hs = (rnv[:, 0:H] + rnv[:, H:2 * H]
              + rnv[:, 2 * H:3 * H] + rnv[:, 3 * H:4 * H])
        o_ref[...] = jnp.tanh(_dot(fr_ref[...], w1_ref[...])
                              + _dot(hs, w2_ref[...]) + b_ref[...])

    return pl.pallas_call(
        body,
        grid=(1,),
        in_specs=[_full((nr, H)), _full((nr, 4 * H)), _full((H, H)),
                  _full((H, H)), _full((1, H))],
        out_specs=_full((nr, H)),
        out_shape=jax.ShapeDtypeStruct((nr, H), F32),
    )(fr, rn, w1, w2, b.reshape(1, H))


def _ceil_to(x, m):
    return (x + m - 1) // m * m


def _pad_rows(x, n):
    return jnp.pad(x, ((0, n - x.shape[0]),) + ((0, 0),) * (x.ndim - 1))


def _pad_idx(idx, n):
    return jnp.pad(idx.astype(jnp.int32), (0, n - idx.shape[0]))


def _gate_weights(p):
    wcat = jnp.concatenate(
        [p['Wi'][0], p['Wog'][0], p['Wu'][0], p['Wf'][0]], axis=1)
    bcat = jnp.concatenate(
        [p['Wi'][1], p['Wog'][1], p['Wu'][1], p['Wf'][1]])
    return wcat, bcat


def _pad_idx2(idx, n):
    return jnp.pad(idx.astype(jnp.int32), ((0, 0), (0, n - idx.shape[1])))


def _msg_pass(xg, pos, ptab, bg_t, wcat, din, mp, bond=None, btab=None):
    wfh = wcat[din:, 3 * H:4 * H]
    wh3 = wcat[din:, 0:3 * H]
    hgc0 = _k_d0(xg, pos, ptab, wfh, bond, btab)
    hgcn = _sc_gather_pipe([hgc0], bg_t, 3 * H)[0].reshape(4, mp, 3 * H)
    return _k_d1(xg, pos, ptab, hgcn, wh3, bond, btab)


def kernel(params, tree_fnode, tree_fmess, tree_agraph, tree_bgraph,
           tree_cgraph, roots, graph_fnode, graph_fmess, graph_agraph,
           graph_bgraph):
    p = params
    nt, mt = tree_fnode.shape[0], tree_fmess.shape[0]
    ng, mg = graph_fnode.shape[0], graph_fmess.shape[0]
    cl = tree_cgraph.shape[1]
    nroot = roots.shape[0]
    ntp, mtp = _ceil_to(nt, 2048), _ceil_to(mt, 2048)
    ngp, mgp = _ceil_to(ng, 2048), _ceil_to(mg, 2048)

    fnode_g_p = _pad_idx(graph_fnode, ngp)
    src_g = _pad_idx(graph_fmess[:, 0], mgp)
    bond_g = _pad_idx(graph_fmess[:, 2], mgp)
    pos_g = _pad_idx(graph_fmess[:, 3], mgp)
    bg_g = _pad_idx2(graph_bgraph.T, mgp)
    ag_g = _pad_idx2(graph_agraph.T, ngp)
    src_t = _pad_idx(tree_fmess[:, 0], mtp)
    pos_t = _pad_idx(tree_fmess[:, 2], mtp)
    bg_t = _pad_idx2(tree_bgraph.T, mtp)
    ag_t = _pad_idx2(tree_agraph.T, ntp)
    cg_t = _pad_idx2(tree_cgraph.T, ntp)
    tf0 = _pad_idx(tree_fnode[:, 0], ntp)
    tf1 = _pad_idx(tree_fnode[:, 1], ntp)
    tf2 = _pad_idx(tree_fnode[:, 2], ntp)

    wcat_g, bcat_g = _gate_weights(p['enc_graph'])
    atom_tab = wcat_g[0:ATOM_SIZE]
    bond_tab = _pad_rows(wcat_g[ATOM_SIZE:ATOM_SIZE + NBOND], 8)
    pos_tab_g = _pad_rows(wcat_g[ATOM_SIZE + NBOND:ATOM_SIZE + NBOND + MAX_POS]
                          + bcat_g[None, :], 24)
    atom_rows = _sc_gather(atom_tab, fnode_g_p)
    xg_g = _sc_gather(atom_rows, src_g)
    h1_g = _msg_pass(xg_g, pos_g, pos_tab_g, bg_g, wcat_g,
                     ATOM_SIZE + NBOND + MAX_POS, mgp, bond_g, bond_tab)
    wo_g, bo_g = p['enc_graph']['Wo']
    p_g = _sc_gather(wo_g[0:ATOM_SIZE], fnode_g_p)
    nn_g = _sc_gather_sum(h1_g, ag_g)
    hatom_p = _k_node(p_g, nn_g, wo_g[ATOM_SIZE:], bo_g, True)

    wcat_b, bcat_b = _gate_weights(p['enc_bond'])
    hnode_b = _sc_gather(p['E_l'], tf2)
    t_b = _k_mm(hnode_b, wcat_b[0:H], bcat_b)
    h1_b = _msg_pass(_sc_gather(t_b, src_t), pos_t,
                     _pad_rows(wcat_b[H:H + MAX_POS], 24),
                     bg_t, wcat_b, H + MAX_POS, mtp)
    wo_b, bo_b = p['enc_bond']['Wo']
    p_b = _k_mm(hnode_b, wo_b[0:H], jnp.zeros((H,), F32))
    nn_b = _sc_gather_sum(h1_b, ag_t)
    hbond_p = _k_node(p_b, nn_b, wo_b[H:], bo_b, True)

    w_i, b_i = p['W_i']
    ei_tab = _k_mm(_pad_rows(p['E_i'], _ceil_to(p['E_i'].shape[0], BM)),
                   w_i[0:H], jnp.zeros((H,), F32))
    eg_f = _sc_gather(ei_tab, tf1)
    cl_sum = _sc_gather_sum(hatom_p, cg_t)
    hnode_f = _k_node(eg_f, cl_sum, w_i[H:], b_i, False)
    wcat_f, bcat_f = _gate_weights(p['enc_frag'])
    t_f = _k_mm(hnode_f, wcat_f[0:H], bcat_f)
    h1_f = _msg_pass(_sc_gather(t_f, src_t), pos_t,
                     _pad_rows(wcat_f[H:H + MAX_POS], 24),
                     bg_t, wcat_f, H + MAX_POS, mtp)
    wo_f, bo_f = p['enc_frag']['Wo']
    p_f = _k_mm(hnode_f, wo_f[0:H], jnp.zeros((H,), F32))
    nn_f = _sc_gather_sum(h1_f, ag_t)
    hinter_p = _k_node(p_f, nn_f, wo_f[H:], bo_f, True)

    w_c, b_c = p['W_c']
    ec_tab = _k_mm(_pad_rows(p['E_c'], _ceil_to(p['E_c'].shape[0], BM)),
                   w_c[0:H], jnp.zeros((H,), F32))
    eg_c = _sc_gather(ec_tab, tf0)
    hnode_c = _k_hc(eg_c, hinter_p, hbond_p, w_c[H:2 * H], w_c[2 * H:], b_c)
    wcat_c, bcat_c = _gate_weights(p['enc_inter'])
    t_c = _k_mm(hnode_c, wcat_c[0:H], bcat_c)
    h1_c = _msg_pass(_sc_gather(t_c, src_t), pos_t,
                     _pad_rows(wcat_c[H:H + MAX_POS], 24),
                     bg_t, wcat_c, H + MAX_POS, mtp)
    wo_c, bo_c = p['enc_inter']['Wo']
    p_c = _k_mm(hnode_c, wo_c[0:H], jnp.zeros((H,), F32))
    nn_c = _sc_gather_sum(h1_c, ag_t)
    hnode_p = _k_node(p_c, nn_c, wo_c[H:], bo_c, True)

    wr, br = p['W_root']
    roots_p = _pad_idx(roots, 256)
    fr = _sc_gather(hnode_c, roots_p)[:nroot]
    agr = _pad_idx(jnp.take(tree_agraph, roots, axis=0).reshape(-1), 256)
    rn = _sc_gather(h1_c, agr)[:nroot * 4].reshape(nroot, 4 * H)
    hroot = _k_root(fr, rn, wr[0:H], wr[H:], br)

    return (hroot, hnode_p[:nt], hinter_p[:nt], hbond_p[:nt], hatom_p[:ng])


# speedup vs baseline: 2.1763x; 1.0010x over previous
import functools

import jax
import jax.numpy as jnp
from jax import lax
from jax.experimental import pallas as pl
from jax.experimental.pallas import tpu as pltpu, tpu_sc as plsc

H = 128
MAX_POS = 20
ATOM_SIZE = 40
NBOND = 4
NW = 32
BM = 512
F32 = jnp.float32


def _pick_chunk2(bpw, d, a_dim, ntab):
    cap = max(8, min(bpw, 470_000 // (2 * ntab * a_dim * d * 4)))
    best = 8
    for c in range(8, cap + 1, 8):
        if bpw % c == 0:
            best = c
    return best


def _sc_gather_pipe(tables, idx_t, d):
    a_dim, mp = idx_t.shape
    ntab = len(tables)
    bpw = mp // NW
    chunk = _pick_chunk2(bpw, d, a_dim, ntab)
    nloop = bpw // chunk
    k_idx = a_dim * chunk
    idx_p = idx_t.reshape(a_dim, NW, nloop, chunk).transpose(1, 2, 0, 3).reshape(-1)
    mesh = plsc.VectorSubcoreMesh(core_axis_name="c", subcore_axis_name="s")
    shp = jax.ShapeDtypeStruct((a_dim * mp, d), F32)

    @functools.partial(
        pl.kernel,
        out_type=tuple(shp for _ in range(ntab)),
        mesh=mesh,
        scratch_types=[
            pltpu.VMEM((k_idx,), jnp.int32),
            pltpu.VMEM((k_idx,), jnp.int32),
            pltpu.VMEM((ntab * k_idx, d), F32),
            pltpu.VMEM((ntab * k_idx, d), F32),
            pltpu.SemaphoreType.DMA,
            pltpu.SemaphoreType.DMA,
            pltpu.SemaphoreType.DMA,
            pltpu.SemaphoreType.DMA,
            pltpu.SemaphoreType.DMA,
        ],
    )
    def k(*refs):
        tabs = refs[:ntab]
        idx_hbm = refs[ntab]
        outs = refs[ntab + 1:ntab + 1 + ntab]
        (i0, i1, r0, r1, is0, is1, gsem, ss0, ss1) = refs[ntab + 1 + ntab:]
        idxb, rowsb, isems, ssems = [i0, i1], [r0, r1], [is0, is1], [ss0, ss1]
        wid = lax.axis_index("s") * 2 + lax.axis_index("c")
        ibase = wid * nloop * k_idx
        obase = wid * bpw

        def idx_src(j):
            return idx_hbm.at[pl.ds(ibase + j * k_idx, k_idx)]

        def out_reg(t, a, j):
            return outs[t].at[pl.ds(a * mp + obase + j * chunk, chunk)]

        def rslice(rb, t, a):
            return rb.at[pl.ds((t * a_dim + a) * chunk, chunk)]

        def _when(cond, fn):
            if isinstance(cond, bool):
                if cond:
                    fn()
            else:
                pl.when(cond)(fn)

        def step(j, b):
            def drain_prev():
                for t in range(ntab):
                    for a in range(a_dim):
                        pltpu.make_async_copy(
                            rslice(rowsb[b], t, a), out_reg(t, a, j - 2),
                            ssems[b]).wait()
            _when(j >= 2, drain_prev)
            pltpu.make_async_copy(idx_src(j), idxb[b], isems[b]).wait()
            descs = [
                pltpu.async_copy(
                    tabs[t].at[idxb[b]],
                    rowsb[b].at[pl.ds(t * k_idx, k_idx)], gsem)
                for t in range(ntab)
            ]
            for de in descs:
                de.wait()

            def prefetch():
                pltpu.async_copy(idx_src(j + 2), idxb[b], isems[b])
            _when(j + 2 < nloop, prefetch)
            for t in range(ntab):
                for a in range(a_dim):
                    pltpu.async_copy(rslice(rowsb[b], t, a), out_reg(t, a, j),
                                     ssems[b])

        pltpu.async_copy(idx_src(0), idxb[0], isems[0])
        if nloop > 1:
            pltpu.async_copy(idx_src(1), idxb[1], isems[1])

        def body2(j0, carry):
            step(j0 * 2, 0)
            step(j0 * 2 + 1, 1)
            return carry

        lax.fori_loop(0, nloop // 2, body2, 0)
        if nloop % 2:
            step(nloop - 1, (nloop - 1) % 2)
        for jl in range(max(0, nloop - 2), nloop):
            b = jl % 2
            for t in range(ntab):
                for a in range(a_dim):
                    pltpu.make_async_copy(
                        rslice(rowsb[b], t, a), out_reg(t, a, jl),
                        ssems[b]).wait()

    out = k(*tables, idx_p)
    return tuple(out) if isinstance(out, (list, tuple)) else (out,)


def _sc_gather_sum(table, idx_t):
    a_dim, mp = idx_t.shape
    bpw = mp // NW
    chunk = _pick_chunk2(bpw, H, a_dim, 1)
    nloop = bpw // chunk
    k_idx = a_dim * chunk
    idx_p = idx_t.reshape(a_dim, NW, nloop, chunk).transpose(1, 2, 0, 3).reshape(-1)
    mesh = plsc.VectorSubcoreMesh(core_axis_name="c", subcore_axis_name="s")

    @functools.partial(
        pl.kernel,
        out_type=jax.ShapeDtypeStruct((mp, H), F32),
        mesh=mesh,
        scratch_types=[
            pltpu.VMEM((k_idx,), jnp.int32),
            pltpu.VMEM((k_idx,), jnp.int32),
            pltpu.VMEM((k_idx, H), F32),
            pltpu.VMEM((k_idx, H), F32),
            pltpu.VMEM((chunk, H), F32),
            pltpu.VMEM((chunk, H), F32),
            pltpu.SemaphoreType.DMA,
            pltpu.SemaphoreType.DMA,
            pltpu.SemaphoreType.DMA,
            pltpu.SemaphoreType.DMA,
            pltpu.SemaphoreType.DMA,
        ],
    )
    def k(tab_hbm, idx_hbm, out_hbm, i0, i1, r0, r1, h0, h1,
          is0, is1, gsem, ss0, ss1):
        idxb, rowsb, hsb = [i0, i1], [r0, r1], [h0, h1]
        isems, ssems = [is0, is1], [ss0, ss1]
        wid = lax.axis_index("s") * 2 + lax.axis_index("c")
        ibase = wid * nloop * k_idx
        obase = wid * bpw

        def idx_src(j):
            return idx_hbm.at[pl.ds(ibase + j * k_idx, k_idx)]

        def out_reg(j):
            return out_hbm.at[pl.ds(obase + j * chunk, chunk)]

        def _when(cond, fn):
            if isinstance(cond, bool):
                if cond:
                    fn()
            else:
                pl.when(cond)(fn)

        def step(j, b):
            def drain_prev():
                pltpu.make_async_copy(hsb[b], out_reg(j - 2), ssems[b]).wait()
            _when(j >= 2, drain_prev)
            pltpu.make_async_copy(idx_src(j), idxb[b], isems[b]).wait()
            pltpu.async_copy(tab_hbm.at[idxb[b]], rowsb[b], gsem).wait()

            def prefetch():
                pltpu.async_copy(idx_src(j + 2), idxb[b], isems[b])
            _when(j + 2 < nloop, prefetch)

            def red(m, carry):
                for v in range(H // 16):
                    acc = rowsb[b][m, pl.ds(v * 16, 16)]
                    for a in range(1, a_dim):
                        acc = acc + rowsb[b][a * chunk + m, pl.ds(v * 16, 16)]
                    hsb[b][m, pl.ds(v * 16, 16)] = acc
                return carry

            lax.fori_loop(0, chunk, red, 0)
            pltpu.async_copy(hsb[b], out_reg(j), ssems[b])

        pltpu.async_copy(idx_src(0), idxb[0], isems[0])
        if nloop > 1:
            pltpu.async_copy(idx_src(1), idxb[1], isems[1])

        def body2(j0, carry):
            step(j0 * 2, 0)
            step(j0 * 2 + 1, 1)
            return carry

        lax.fori_loop(0, nloop // 2, body2, 0)
        if nloop % 2:
            step(nloop - 1, (nloop - 1) % 2)
        for jl in range(max(0, nloop - 2), nloop):
            pltpu.make_async_copy(hsb[jl % 2], out_reg(jl), ssems[jl % 2]).wait()

    return k(table, idx_p)


def _sc_gather(table, idx):
    return _sc_gather_pipe([table], idx[None, :], table.shape[1])[0]


def _row(d):
    return pl.BlockSpec((BM, d), lambda i: (i, 0))


def _full(shape):
    return pl.BlockSpec(shape, lambda i: tuple(0 for _ in shape))


def _rowmask(x):
    rows = pl.program_id(0) * BM + lax.broadcasted_iota(jnp.int32, (BM, 1), 0)
    return x * (rows != 0).astype(F32)


def _dot(a, b):
    return jnp.dot(a, b, preferred_element_type=F32)


def _sigm(x):
    return jax.nn.sigmoid(x)


def _k_mm(x, w, b, relu=False):
    m, kdim = x.shape
    n = w.shape[1]

    def body(x_ref, w_ref, b_ref, o_ref):
        acc = _dot(x_ref[...], w_ref[...]) + b_ref[...]
        o_ref[...] = jnp.maximum(acc, 0.0) if relu else acc

    return pl.pallas_call(
        body,
        grid=(m // BM,),
        in_specs=[_row(kdim), _full((kdim, n)), _full((1, n))],
        out_specs=_row(n),
        out_shape=jax.ShapeDtypeStruct((m, n), F32),
    )(x, w, b.reshape(1, n))


def _xparts(xg, pos_ref, ptab_ref, bond_ref, btab_ref, pw, bw):
    oh_p = (pos_ref[...] == lax.broadcasted_iota(jnp.int32, (BM, pw), 1)
            ).astype(F32)
    x = xg + _dot(oh_p, ptab_ref[...])
    if btab_ref is not None:
        oh_b = (bond_ref[...] == lax.broadcasted_iota(jnp.int32, (BM, bw), 1)
                ).astype(F32)
        x = x + _dot(oh_b, btab_ref[...])
    return x


def _k_d0(xg, pos, ptab, wfh, bond=None, btab=None):
    m = xg.shape[0]
    pw = ptab.shape[0]
    bw = btab.shape[0] if btab is not None else 0

    def body(*refs):
        if btab is None:
            xg_ref, pos_ref, ptab_ref, wfh_ref, hgc_ref = refs
            bond_ref = btab_ref = None
        else:
            (xg_ref, pos_ref, bond_ref, ptab_ref, btab_ref, wfh_ref,
             hgc_ref) = refs
        xv = _xparts(xg_ref[...], pos_ref, ptab_ref, bond_ref, btab_ref, pw, bw)
        i = _sigm(xv[:, 0:H])
        o = _sigm(xv[:, H:2 * H])
        u = jnp.tanh(xv[:, 2 * H:3 * H])
        c = _rowmask(i * u)
        h = _rowmask(o * jnp.tanh(c))
        hgc_ref[...] = jnp.concatenate([h, _dot(h, wfh_ref[...]), c], axis=-1)

    specs = [_row(4 * H), pl.BlockSpec((BM, 1), lambda i: (i, 0))]
    args = [xg, pos.reshape(m, 1)]
    if btab is not None:
        specs.append(pl.BlockSpec((BM, 1), lambda i: (i, 0)))
        args.append(bond.reshape(m, 1))
    specs.append(_full(ptab.shape))
    args.append(ptab)
    if btab is not None:
        specs.append(_full(btab.shape))
        args.append(btab)
    specs.append(_full((H, H)))
    args.append(wfh)

    return pl.pallas_call(
        body,
        grid=(m // BM,),
        in_specs=specs,
        out_specs=_row(3 * H),
        out_shape=jax.ShapeDtypeStruct((m, 3 * H), F32),
    )(*args)


def _k_d1(xg, pos, ptab, hgcn, wh3, bond=None, btab=None):
    m = xg.shape[0]
    pw = ptab.shape[0]
    bw = btab.shape[0] if btab is not None else 0
    nbr = pl.BlockSpec((4, BM, 3 * H), lambda i: (0, i, 0))

    def body(*refs):
        if btab is None:
            (xg_ref, pos_ref, ptab_ref, hgcn_ref, wh3_ref, h_ref) = refs
            bond_ref = btab_ref = None
        else:
            (xg_ref, pos_ref, bond_ref, ptab_ref, btab_ref, hgcn_ref,
             wh3_ref, h_ref) = refs
        xv = _xparts(xg_ref[...], pos_ref, ptab_ref, bond_ref, btab_ref, pw, bw)
        hs = (hgcn_ref[0, :, 0:H] + hgcn_ref[1, :, 0:H]
              + hgcn_ref[2, :, 0:H] + hgcn_ref[3, :, 0:H])
        z = _dot(hs, wh3_ref[...])
        i = _sigm(xv[:, 0:H] + z[:, 0:H])
        o = _sigm(xv[:, H:2 * H] + z[:, H:2 * H])
        u = jnp.tanh(xv[:, 2 * H:3 * H] + z[:, 2 * H:3 * H])
        xf = xv[:, 3 * H:4 * H]
        fc = jnp.zeros((BM, H), F32)
        for a in range(4):
            fc = fc + (_sigm(xf + hgcn_ref[a, :, H:2 * H])
                       * hgcn_ref[a, :, 2 * H:3 * H])
        c = i * u + fc
        h_ref[...] = _rowmask(o * jnp.tanh(c))

    specs = [_row(4 * H), pl.BlockSpec((BM, 1), lambda i: (i, 0))]
    args = [xg, pos.reshape(m, 1)]
    if btab is not None:
        specs.append(pl.BlockSpec((BM, 1), lambda i: (i, 0)))
        args.append(bond.reshape(m, 1))
    specs.append(_full(ptab.shape))
    args.append(ptab)
    if btab is not None:
        specs.append(_full(btab.shape))
        args.append(btab)
    specs += [nbr, _full((H, 3 * H))]
    args += [hgcn, wh3]

    return pl.pallas_call(
        body,
        grid=(m // BM,),
        in_specs=specs,
        out_specs=_row(H),
        out_shape=jax.ShapeDtypeStruct((m, H), F32),
    )(*args)


def _k_node(p, hs, w, b, mask0):
    m = p.shape[0]

    def body(p_ref, hs_ref, w_ref, b_ref, o_ref):
        out = jnp.maximum(
            p_ref[...] + _dot(hs_ref[...], w_ref[...]) + b_ref[...], 0.0)
        o_ref[...] = _rowmask(out) if mask0 else out

    return pl.pallas_call(
        body,
        grid=(m // BM,),
        in_specs=[_row(H), _row(H), _full((H, H)), _full((1, H))],
        out_specs=_row(H),
        out_shape=jax.ShapeDtypeStruct((m, H), F32),
    )(p, hs, w, b.reshape(1, H))


def _k_hc(eg, hi, hb, w1, w2, b):
    m = eg.shape[0]

    def body(eg_ref, hi_ref, hb_ref, w1_ref, w2_ref, b_ref, o_ref):
        o_ref[...] = jnp.maximum(
            eg_ref[...] + _dot(hi_ref[...], w1_ref[...])
            + _dot(hb_ref[...], w2_ref[...]) + b_ref[...], 0.0)

    return pl.pallas_call(
        body,
        grid=(m // BM,),
        in_specs=[_row(H), _row(H), _row(H), _full((H, H)), _full((H, H)),
                  _full((1, H))],
        out_specs=_row(H),
        out_shape=jax.ShapeDtypeStruct((m, H), F32),
    )(eg, hi, hb, w1, w2, b.reshape(1, H))


def _k_root(fr, rn, w1, w2, b):
    nr = fr.shape[0]

    def body(fr_ref, rn_ref, w1_ref, w2_ref, b_ref, o_ref):
        rnv = rn_ref[...]
        hs = (rnv[:, 0:H] + rnv[:, H:2 * H]
              + rnv[:, 2 * H:3 * H] + rnv[:, 3 * H:4 * H])
        o_ref[...] = jnp.tanh(_dot(fr_ref[...], w1_ref[...])
                              + _dot(hs, w2_ref[...]) + b_ref[...])

    return pl.pallas_call(
        body,
        grid=(1,),
        in_specs=[_full((nr, H)), _full((nr, 4 * H)), _full((H, H)),
                  _full((H, H)), _full((1, H))],
        out_specs=_full((nr, H)),
        out_shape=jax.ShapeDtypeStruct((nr, H), F32),
    )(fr, rn, w1, w2, b.reshape(1, H))


def _ceil_to(x, m):
    return (x + m - 1) // m * m


def _pad_rows(x, n):
    return jnp.pad(x, ((0, n - x.shape[0]),) + ((0, 0),) * (x.ndim - 1))


def _pad_idx(idx, n):
    return jnp.pad(idx.astype(jnp.int32), (0, n - idx.shape[0]))


def _gate_weights(p):
    wcat = jnp.concatenate(
        [p['Wi'][0], p['Wog'][0], p['Wu'][0], p['Wf'][0]], axis=1)
    bcat = jnp.concatenate(
        [p['Wi'][1], p['Wog'][1], p['Wu'][1], p['Wf'][1]])
    return wcat, bcat


def _pad_idx2(idx, n):
    return jnp.pad(idx.astype(jnp.int32), ((0, 0), (0, n - idx.shape[1])))


def _msg_pass(xg, pos, ptab, bg_t, wcat, din, mp, bond=None, btab=None):
    wfh = wcat[din:, 3 * H:4 * H]
    wh3 = wcat[din:, 0:3 * H]
    hgc0 = _k_d0(xg, pos, ptab, wfh, bond, btab)
    hgcn = _sc_gather_pipe([hgc0], bg_t, 3 * H)[0].reshape(4, mp, 3 * H)
    return _k_d1(xg, pos, ptab, hgcn, wh3, bond, btab)


def kernel(params, tree_fnode, tree_fmess, tree_agraph, tree_bgraph,
           tree_cgraph, roots, graph_fnode, graph_fmess, graph_agraph,
           graph_bgraph):
    p = params
    nt, mt = tree_fnode.shape[0], tree_fmess.shape[0]
    ng, mg = graph_fnode.shape[0], graph_fmess.shape[0]
    cl = tree_cgraph.shape[1]
    nroot = roots.shape[0]
    ntp, mtp = _ceil_to(nt, 2048), _ceil_to(mt, 2048)
    ngp, mgp = _ceil_to(ng, 2048), _ceil_to(mg, 2048)

    fnode_g_p = _pad_idx(graph_fnode, ngp)
    src_g = _pad_idx(graph_fmess[:, 0], mgp)
    bond_g = _pad_idx(graph_fmess[:, 2], mgp)
    pos_g = _pad_idx(graph_fmess[:, 3], mgp)
    bg_g = _pad_idx2(graph_bgraph.T, mgp)
    ag_g = _pad_idx2(graph_agraph.T, ngp)
    src_t = _pad_idx(tree_fmess[:, 0], mtp)
    pos_t = _pad_idx(tree_fmess[:, 2], mtp)
    bg_t = _pad_idx2(tree_bgraph.T, mtp)
    ag_t = _pad_idx2(tree_agraph.T, ntp)
    cg_t = _pad_idx2(tree_cgraph.T, ntp)
    tf0 = _pad_idx(tree_fnode[:, 0], ntp)
    tf1 = _pad_idx(tree_fnode[:, 1], ntp)
    tf2 = _pad_idx(tree_fnode[:, 2], ntp)

    wcat_g, bcat_g = _gate_weights(p['enc_graph'])
    atom_tab = wcat_g[0:ATOM_SIZE]
    bond_tab = _pad_rows(wcat_g[ATOM_SIZE:ATOM_SIZE + NBOND], 8)
    pos_tab_g = _pad_rows(wcat_g[ATOM_SIZE + NBOND:ATOM_SIZE + NBOND + MAX_POS]
                          + bcat_g[None, :], 24)
    atom_rows = _sc_gather(atom_tab, fnode_g_p)
    xg_g = _sc_gather(atom_rows, src_g)
    h1_g = _msg_pass(xg_g, pos_g, pos_tab_g, bg_g, wcat_g,
                     ATOM_SIZE + NBOND + MAX_POS, mgp, bond_g, bond_tab)
    wo_g, bo_g = p['enc_graph']['Wo']
    p_g = _sc_gather(wo_g[0:ATOM_SIZE], fnode_g_p)
    nn_g = _sc_gather_sum(h1_g, ag_g)
    hatom_p = _k_node(p_g, nn_g, wo_g[ATOM_SIZE:], bo_g, True)

    wcat_b, bcat_b = _gate_weights(p['enc_bond'])
    hnode_b = _sc_gather(p['E_l'], tf2)
    t_b = _k_mm(hnode_b, wcat_b[0:H], bcat_b)
    h1_b = _msg_pass(_sc_gather(t_b, src_t), pos_t,
                     _pad_rows(wcat_b[H:H + MAX_POS], 24),
                     bg_t, wcat_b, H + MAX_POS, mtp)
    wo_b, bo_b = p['enc_bond']['Wo']
    p_b = _k_mm(hnode_b, wo_b[0:H], jnp.zeros((H,), F32))
    nn_b = _sc_gather_sum(h1_b, ag_t)
    hbond_p = _k_node(p_b, nn_b, wo_b[H:], bo_b, True)

    w_i, b_i = p['W_i']
    ei_tab = _k_mm(_pad_rows(p['E_i'], _ceil_to(p['E_i'].shape[0], BM)),
                   w_i[0:H], jnp.zeros((H,), F32))
    eg_f = _sc_gather(ei_tab, tf1)
    cl_sum = _sc_gather_sum(hatom_p, cg_t)
    hnode_f = _k_node(eg_f, cl_sum, w_i[H:], b_i, False)
    wcat_f, bcat_f = _gate_weights(p['enc_frag'])
    t_f = _k_mm(hnode_f, wcat_f[0:H], bcat_f)
    h1_f = _msg_pass(_sc_gather(t_f, src_t), pos_t,
                     _pad_rows(wcat_f[H:H + MAX_POS], 24),
                     bg_t, wcat_f, H + MAX_POS, mtp)
    wo_f, bo_f = p['enc_frag']['Wo']
    p_f = _k_mm(hnode_f, wo_f[0:H], jnp.zeros((H,), F32))
    nn_f = _sc_gather_sum(h1_f, ag_t)
    hinter_p = _k_node(p_f, nn_f, wo_f[H:], bo_f, True)

    w_c, b_c = p['W_c']
    ec_tab = _k_mm(_pad_rows(p['E_c'], _ceil_to(p['E_c'].shape[0], BM)),
                   w_c[0:H], jnp.zeros((H,), F32))
    eg_c = _sc_gather(ec_tab, tf0)
    hnode_c = _k_hc(eg_c, hinter_p, hbond_p, w_c[H:2 * H], w_c[2 * H:], b_c)
    wcat_c, bcat_c = _gate_weights(p['enc_inter'])
    t_c = _k_mm(hnode_c, wcat_c[0:H], bcat_c)
    h1_c = _msg_pass(_sc_gather(t_c, src_t), pos_t,
                     _pad_rows(wcat_c[H:H + MAX_POS], 24),
                     bg_t, wcat_c, H + MAX_POS, mtp)
    wo_c, bo_c = p['enc_inter']['Wo']
    p_c = _k_mm(hnode_c, wo_c[0:H], jnp.zeros((H,), F32))
    nn_c = _sc_gather_sum(h1_c, ag_t)
    hnode_p = _k_node(p_c, nn_c, wo_c[H:], bo_c, True)

    wr, br = p['W_root']
    roots_p = _pad_idx(roots, 256)
    fr = _sc_gather(hnode_c, roots_p)[:nroot]
    agr = _pad_idx(jnp.take(tree_agraph, roots, axis=0).reshape(-1), 256)
    rn = _sc_gather(h1_c, agr)[:nroot * 4].reshape(nroot, 4 * H)
    hroot = _k_root(fr, rn, wr[0:H], wr[H:], br)

    return (hroot, hnode_p[:nt], hinter_p[:nt], hbond_p[:nt], hatom_p[:ng])
